# baseline (speedup 1.0000x reference)
import functools

import jax
import jax.numpy as jnp
from jax import lax
from jax.experimental import pallas as pl
from jax.experimental.pallas import tpu as pltpu
from jax.experimental.pallas import tpu_sc as plsc

L = 4
N = 10000
E = 160000
BN = 1000
NB = N // BN

SC_C = 128
SC_J = 40
E_PAD = 32 * SC_J * SC_C
IDX_ROWS = E_PAD // SC_C
N_ACC = N + 128


def _silu(x):
    return x * jax.nn.sigmoid(x)


def _ln(h, g, b):
    m = jnp.mean(h, axis=-1, keepdims=True)
    d = h - m
    v = jnp.mean(d * d, axis=-1, keepdims=True)
    return d * lax.rsqrt(v + 1e-5) * g + b


def _elu1(z):
    return jnp.where(z > 0, z + 1.0, jnp.exp(jnp.minimum(z, 0.0)))


def _full_spec(shape):
    nd = len(shape)
    return pl.BlockSpec(shape, lambda n, _nd=nd: (0,) * _nd)


def _xspec(d):
    return pl.BlockSpec((L, BN, d), lambda n: (0, n, 0))


def _k0_body(opc_ref, normal_ref, pe_ref, emb_ref, win_e_ref, win_n_ref,
             bin_ref, gin_ref, bein_ref, wpe_ref, bpe_ref,
             x_ref, peemb_ref):
    iota = lax.broadcasted_iota(jnp.int32, (BN, 128), 1).astype(jnp.float32)
    for l in range(L):
        opc = opc_ref[l]
        oh = jnp.where(opc == iota, 1.0, 0.0)
        emb = jnp.dot(oh, emb_ref[...], preferred_element_type=jnp.float32)
        pre = (emb @ win_e_ref[...] + normal_ref[l] @ win_n_ref[...]
               + bin_ref[...])
        x_ref[l] = _ln(_silu(pre), gin_ref[...], bein_ref[...])
        peemb_ref[l] = jnp.tanh(pe_ref[l] @ wpe_ref[...] + bpe_ref[...])


def _call_k0(opc, normal, pe_feat, p):
    win = p['W_in']
    return pl.pallas_call(
        _k0_body,
        grid=(NB,),
        in_specs=[
            pl.BlockSpec((L, BN, 1), lambda n: (0, n, 0)),
            pl.BlockSpec((L, BN, 127), lambda n: (0, n, 0)),
            _xspec(16),
            _full_spec((128, 32)),
            _full_spec((32, 128)),
            _full_spec((127, 128)),
            _full_spec((1, 128)),
            _full_spec((1, 128)),
            _full_spec((1, 128)),
            _full_spec((16, 32)),
            _full_spec((1, 32)),
        ],
        out_specs=[_xspec(128), _xspec(32)],
        out_shape=[
            jax.ShapeDtypeStruct((L, N, 128), jnp.float32),
            jax.ShapeDtypeStruct((L, N, 32), jnp.float32),
        ],
    )(opc, normal, pe_feat, p['emb'], win[:32], win[32:],
      p['b_in'][None], p['g_in'][None], p['be_in'][None],
      p['W_pe'], p['b_pe'][None])


def _k1_body(pe_ref, x_ref, w_pe_ref, w_x_ref, b_ref, wi_pe_ref, wi_x_ref,
             bi_ref, wo_pe_ref, wo_x_ref, bo_ref,
             pw_ref, pb_ref, pwi_ref, pbi_ref, pwo_ref, pbo_ref,
             proj_ref, peproj_ref, pin_ref, pout_ref, pepin_ref, pepout_ref):
    for l in range(L):
        pe = pe_ref[l]
        x = x_ref[l]
        proj_ref[l] = pe @ w_pe_ref[...] + x @ w_x_ref[...] + b_ref[...]
        pin_ref[:, l, :] = pe @ wi_pe_ref[...] + x @ wi_x_ref[...] + bi_ref[...]
        pout_ref[:, l, :] = pe @ wo_pe_ref[...] + x @ wo_x_ref[...] + bo_ref[...]
        peproj_ref[l] = pe @ pw_ref[...] + pb_ref[...]
        pepin_ref[:, l, :] = pe @ pwi_ref[...] + pbi_ref[...]
        pepout_ref[:, l, :] = pe @ pwo_ref[...] + pbo_ref[...]


def _call_k1(pe_emb, x, lp):
    f, pp = lp['feat'], lp['pe']
    tab_spec = pl.BlockSpec((BN, L, 32), lambda n: (n, 0, 0))
    tab_shape = jax.ShapeDtypeStruct((N, L, 32), jnp.float32)
    return pl.pallas_call(
        _k1_body,
        grid=(NB,),
        in_specs=[
            _xspec(32),
            _xspec(128),
            _full_spec((32, 128)), _full_spec((128, 128)), _full_spec((1, 128)),
            _full_spec((32, 32)), _full_spec((128, 32)), _full_spec((1, 32)),
            _full_spec((32, 32)), _full_spec((128, 32)), _full_spec((1, 32)),
            _full_spec((32, 32)), _full_spec((1, 32)),
            _full_spec((32, 32)), _full_spec((1, 32)),
            _full_spec((32, 32)), _full_spec((1, 32)),
        ],
        out_specs=[
            _xspec(128), _xspec(32),
            tab_spec, tab_spec, tab_spec, tab_spec,
        ],
        out_shape=[
            jax.ShapeDtypeStruct((L, N, 128), jnp.float32),
            jax.ShapeDtypeStruct((L, N, 32), jnp.float32),
            tab_shape, tab_shape, tab_shape, tab_shape,
        ],
    )(pe_emb, x,
      f['W'][:32], f['W'][32:], f['b'][None],
      f['Wi'][:32], f['Wi'][32:], f['bi'][None],
      f['Wo'][:32], f['Wo'][32:], f['bo'][None],
      pp['W'], pp['b'][None], pp['Wi'], pp['bi'][None],
      pp['Wo'], pp['bo'][None])


def _sc_body(pin_h, pout_h, pepin_h, pepout_h, sg_h, ss_h, dg_h, ds_h,
             zeros_h, inc_h, outg_h, peinc_h, peoutg_h,
             gidx_v, sidx_v, rows0_v, rows1_v, acc_sh,
             gsem0, gsem1, ssem0, ssem1):
    cid = lax.axis_index("c")
    sid = lax.axis_index("s")
    idx_row0 = cid * (IDX_ROWS // 2) + sid * SC_J
    fl0 = sid * 624
    flr = 640
    rows = (rows0_v, rows1_v)
    gsems = (gsem0, gsem1)
    ssems = (ssem0, ssem1)

    def gather(j, b, tab_h):
        pltpu.async_copy(tab_h.at[gidx_v.at[j]], rows[b], gsems[b])

    def scatter(j, b):
        pltpu.async_copy(rows[b], acc_sh.at[sidx_v.at[j]], ssems[b],
                         add=True)

    for tab_h, gh, sh, out_h in (
            (pin_h, sg_h, ds_h, inc_h),
            (pout_h, dg_h, ss_h, outg_h),
            (pepin_h, sg_h, ds_h, peinc_h),
            (pepout_h, dg_h, ss_h, peoutg_h)):
        pltpu.sync_copy(zeros_h.at[pl.ds(fl0, flr)], acc_sh.at[pl.ds(fl0, flr)])
        pltpu.sync_copy(gh.at[pl.ds(idx_row0, SC_J)], gidx_v)
        pltpu.sync_copy(sh.at[pl.ds(idx_row0, SC_J)], sidx_v)
        plsc.subcore_barrier()

        gather(0, 0, tab_h)

        def jbody(i, carry):
            j0 = 2 * i
            j1 = 2 * i + 1
            pltpu.make_async_copy(tab_h.at[gidx_v.at[j0]], rows[0],
                                  gsems[0]).wait()
            scatter(j0, 0)

            @pl.when(i > 0)
            def _():
                pltpu.make_async_copy(rows[1], acc_sh.at[sidx_v.at[j1]],
                                      ssems[1]).wait()

            gather(j1, 1, tab_h)
            pltpu.make_async_copy(tab_h.at[gidx_v.at[j1]], rows[1],
                                  gsems[1]).wait()
            scatter(j1, 1)

            @pl.when(j1 + 1 < SC_J)
            def _():
                pltpu.make_async_copy(rows[0], acc_sh.at[sidx_v.at[j0]],
                                      ssems[0]).wait()
                gather(j1 + 1, 0, tab_h)

            return carry

        lax.fori_loop(0, SC_J // 2, jbody, 0)
        pltpu.make_async_copy(rows[0], acc_sh.at[sidx_v.at[0]],
                              ssems[0]).wait()
        pltpu.make_async_copy(rows[1], acc_sh.at[sidx_v.at[0]],
                              ssems[1]).wait()
        plsc.subcore_barrier()
        pltpu.sync_copy(acc_sh.at[pl.ds(fl0, flr)],
                        out_h.at[cid].at[pl.ds(fl0, flr)])
        plsc.subcore_barrier()


def _call_sc(pin_t, pout_t, pepin_t, pepout_t, sg, ss, dg, ds_, zeros_nd):
    mesh = plsc.VectorSubcoreMesh(core_axis_name="c", subcore_axis_name="s")
    fn = pl.kernel(
        _sc_body,
        out_type=[jax.ShapeDtypeStruct((2, N, 128), jnp.float32)] * 4,
        mesh=mesh,
        scratch_types=[
            pltpu.VMEM((SC_J, SC_C), jnp.int32),
            pltpu.VMEM((SC_J, SC_C), jnp.int32),
            pltpu.VMEM((SC_C, 128), jnp.float32),
            pltpu.VMEM((SC_C, 128), jnp.float32),
            pltpu.VMEM_SHARED((N_ACC, 128), jnp.float32),
            pltpu.SemaphoreType.DMA,
            pltpu.SemaphoreType.DMA,
            pltpu.SemaphoreType.DMA,
            pltpu.SemaphoreType.DMA,
        ],
    )
    return fn(pin_t.reshape(N, 128), pout_t.reshape(N, 128),
              pepin_t.reshape(N, 128), pepout_t.reshape(N, 128),
              sg, ss, dg, ds_, zeros_nd)


def _k2_body(proj_ref, inc_ref, outg_ref, peproj_ref, pei_ref, peo_ref,
             g1_ref, be1_ref, w2_ref, b2_ref, g2_ref, be2_ref,
             pw2_ref, pb2_ref, wk_ref, bk_ref, wv_ref,
             xn_ref, pen_ref, ktv_ref):
    n = pl.program_id(0)
    for l in range(L):
        agg_in = inc_ref[0, :, l, :] + inc_ref[1, :, l, :]
        agg_out = outg_ref[0, :, l, :] + outg_ref[1, :, l, :]
        out = jnp.concatenate([proj_ref[l], agg_in, agg_out], axis=-1)
        h1 = _ln(_silu(out), g1_ref[...], be1_ref[...])
        h2 = h1 @ w2_ref[...] + b2_ref[...]
        xn = _ln(_silu(h2), g2_ref[...], be2_ref[...])
        xn_ref[l] = xn

        pe_in = pei_ref[0, :, l, :] + pei_ref[1, :, l, :]
        pe_out = peo_ref[0, :, l, :] + peo_ref[1, :, l, :]
        pout = jnp.concatenate([peproj_ref[l], pe_in, pe_out], axis=-1)
        pen_ref[l] = jnp.tanh(jnp.tanh(pout) @ pw2_ref[...] + pb2_ref[...])

        k = _elu1(xn @ wk_ref[...] + bk_ref[...])
        v = xn @ wv_ref[...]
        ktv = lax.dot_general(k, v, (((0,), (0,)), ((), ())),
                              preferred_element_type=jnp.float32)

        @pl.when(n == 0)
        def _():
            ktv_ref[l] = ktv

        @pl.when(n != 0)
        def _():
            ktv_ref[l] += ktv


def _call_k2(proj, incP, outgP, peproj, peincP, peoutgP, lp):
    f, pp, ln_ = lp['feat'], lp['pe'], lp['lin']
    part = pl.BlockSpec((2, BN, L, 32), lambda n: (0, n, 0, 0))
    return pl.pallas_call(
        _k2_body,
        grid=(NB,),
        in_specs=[
            _xspec(128), part, part, _xspec(32), part, part,
            _full_spec((1, 192)), _full_spec((1, 192)),
            _full_spec((192, 128)), _full_spec((1, 128)),
            _full_spec((1, 128)), _full_spec((1, 128)),
            _full_spec((96, 32)), _full_spec((1, 32)),
            _full_spec((128, 64)), _full_spec((1, 64)),
            _full_spec((128, 32)),
        ],
        out_specs=[
            _xspec(128), _xspec(32),
            pl.BlockSpec((L, 64, 32), lambda n: (0, 0, 0)),
        ],
        out_shape=[
            jax.ShapeDtypeStruct((L, N, 128), jnp.float32),
            jax.ShapeDtypeStruct((L, N, 32), jnp.float32),
            jax.ShapeDtypeStruct((L, 64, 32), jnp.float32),
        ],
    )(proj, incP.reshape(2, N, L, 32), outgP.reshape(2, N, L, 32),
      peproj, peincP.reshape(2, N, L, 32), peoutgP.reshape(2, N, L, 32),
      f['g1'][None], f['be1'][None], f['W2'], f['b2'][None],
      f['g2'][None], f['be2'][None],
      pp['W2'], pp['b2'][None],
      ln_['Wk'], ln_['bk'][None], ln_['Wv'])


def _k3_body(xn_ref, ktv_ref, wq_ref, bq_ref, wc_x_ref, wc_a_ref, bc_ref,
             x_ref):
    for l in range(L):
        xn = xn_ref[l]
        q = _elu1(xn @ wq_ref[...] + bq_ref[...])
        att = jnp.dot(q, ktv_ref[l], preferred_element_type=jnp.float32)
        x_ref[l] = _silu(xn @ wc_x_ref[...] + att @ wc_a_ref[...]
                         + bc_ref[...])


def _call_k3(x_new, ktv, lp):
    ln_, cb = lp['lin'], lp['comb']
    wc = cb['Wc']
    return pl.pallas_call(
        _k3_body,
        grid=(NB,),
        in_specs=[
            _xspec(128),
            pl.BlockSpec((L, 64, 32), lambda n: (0, 0, 0)),
            _full_spec((128, 64)), _full_spec((1, 64)),
            _full_spec((128, 128)), _full_spec((32, 128)),
            _full_spec((1, 128)),
        ],
        out_specs=_xspec(128),
        out_shape=jax.ShapeDtypeStruct((L, N, 128), jnp.float32),
    )(x_new, ktv, ln_['Wq'], ln_['bq'][None], wc[:128], wc[128:],
      cb['bc'][None])


def kernel(op_code, features, edge_index, lengths, params):
    p = params
    normal = features[..., :127]
    pe_feat = features[..., 127:]
    opc = op_code.astype(jnp.float32)[..., None]

    src, dst = edge_index[0], edge_index[1]
    pad = E_PAD - E
    zpad = jnp.zeros((pad,), jnp.int32)
    npad = N + (jnp.arange(pad, dtype=jnp.int32) % 128)
    sg = jnp.concatenate([src, zpad]).reshape(IDX_ROWS, SC_C)
    ss = jnp.concatenate([src, npad]).reshape(IDX_ROWS, SC_C)
    dg = jnp.concatenate([dst, zpad]).reshape(IDX_ROWS, SC_C)
    ds_ = jnp.concatenate([dst, npad]).reshape(IDX_ROWS, SC_C)
    zeros_nd = jnp.zeros((N, 128), jnp.float32)

    x, pe_emb = _call_k0(opc, normal, pe_feat, p)
    for lp in p['layers']:
        proj, peproj, pin_t, pout_t, pepin_t, pepout_t = _call_k1(pe_emb, x, lp)
        incP, outgP, peincP, peoutgP = _call_sc(
            pin_t, pout_t, pepin_t, pepout_t, sg, ss, dg, ds_, zeros_nd)
        x_new, pe_emb, ktv = _call_k2(
            proj, incP, outgP, peproj, peincP, peoutgP, lp)
        x = _call_k3(x_new, ktv, lp)
    return x


# speedup vs baseline: 1.8535x; 1.8535x over previous
import functools

import jax
import jax.numpy as jnp
from jax import lax
from jax.experimental import pallas as pl
from jax.experimental.pallas import tpu as pltpu
from jax.experimental.pallas import tpu_sc as plsc

L = 4
N = 10000
E = 160000
BN = 1000
NB = N // BN

SC_C = 128
SC_J = 40
E_PAD = 32 * SC_J * SC_C
IDX_ROWS = E_PAD // SC_C
N_ACC = N + 128


def _silu(x):
    return x * jax.nn.sigmoid(x)


def _ln(h, g, b):
    m = jnp.mean(h, axis=-1, keepdims=True)
    d = h - m
    v = jnp.mean(d * d, axis=-1, keepdims=True)
    return d * lax.rsqrt(v + 1e-5) * g + b


def _elu1(z):
    return jnp.where(z > 0, z + 1.0, jnp.exp(jnp.minimum(z, 0.0)))


def _full_spec(shape):
    nd = len(shape)
    return pl.BlockSpec(shape, lambda n, _nd=nd: (0,) * _nd)


def _xspec(d):
    return pl.BlockSpec((L, BN, d), lambda n: (0, n, 0))


def _k0_body(opc_ref, normal_ref, pe_ref, emb_ref, win_e_ref, win_n_ref,
             bin_ref, gin_ref, bein_ref, wpe_ref, bpe_ref,
             x_ref, peemb_ref):
    iota = lax.broadcasted_iota(jnp.int32, (BN, 128), 1).astype(jnp.float32)
    for l in range(L):
        opc = opc_ref[l]
        oh = jnp.where(opc == iota, 1.0, 0.0)
        emb = jnp.dot(oh, emb_ref[...], preferred_element_type=jnp.float32)
        pre = (emb @ win_e_ref[...] + normal_ref[l] @ win_n_ref[...]
               + bin_ref[...])
        x_ref[l] = _ln(_silu(pre), gin_ref[...], bein_ref[...])
        peemb_ref[l] = jnp.tanh(pe_ref[l] @ wpe_ref[...] + bpe_ref[...])


def _call_k0(opc, normal, pe_feat, p):
    win = p['W_in']
    return pl.pallas_call(
        _k0_body,
        grid=(NB,),
        in_specs=[
            pl.BlockSpec((L, BN, 1), lambda n: (0, n, 0)),
            pl.BlockSpec((L, BN, 127), lambda n: (0, n, 0)),
            _xspec(16),
            _full_spec((128, 32)),
            _full_spec((32, 128)),
            _full_spec((127, 128)),
            _full_spec((1, 128)),
            _full_spec((1, 128)),
            _full_spec((1, 128)),
            _full_spec((16, 32)),
            _full_spec((1, 32)),
        ],
        out_specs=[_xspec(128), _xspec(32)],
        out_shape=[
            jax.ShapeDtypeStruct((L, N, 128), jnp.float32),
            jax.ShapeDtypeStruct((L, N, 32), jnp.float32),
        ],
    )(opc, normal, pe_feat, p['emb'], win[:32], win[32:],
      p['b_in'][None], p['g_in'][None], p['be_in'][None],
      p['W_pe'], p['b_pe'][None])


def _k1_body(pe_ref, x_ref, w_pe_ref, w_x_ref, b_ref, wi_pe_ref, wi_x_ref,
             bi_ref, wo_pe_ref, wo_x_ref, bo_ref,
             pw_ref, pb_ref, pwi_ref, pbi_ref, pwo_ref, pbo_ref,
             proj_ref, peproj_ref, pin_ref, pout_ref, pepin_ref, pepout_ref):
    for l in range(L):
        pe = pe_ref[l]
        x = x_ref[l]
        proj_ref[l] = pe @ w_pe_ref[...] + x @ w_x_ref[...] + b_ref[...]
        pin_ref[:, l, :] = pe @ wi_pe_ref[...] + x @ wi_x_ref[...] + bi_ref[...]
        pout_ref[:, l, :] = pe @ wo_pe_ref[...] + x @ wo_x_ref[...] + bo_ref[...]
        peproj_ref[l] = pe @ pw_ref[...] + pb_ref[...]
        pepin_ref[:, l, :] = pe @ pwi_ref[...] + pbi_ref[...]
        pepout_ref[:, l, :] = pe @ pwo_ref[...] + pbo_ref[...]


def _call_k1(pe_emb, x, lp):
    f, pp = lp['feat'], lp['pe']
    tab_spec = pl.BlockSpec((BN, L, 32), lambda n: (n, 0, 0))
    tab_shape = jax.ShapeDtypeStruct((N, L, 32), jnp.float32)
    return pl.pallas_call(
        _k1_body,
        grid=(NB,),
        in_specs=[
            _xspec(32),
            _xspec(128),
            _full_spec((32, 128)), _full_spec((128, 128)), _full_spec((1, 128)),
            _full_spec((32, 32)), _full_spec((128, 32)), _full_spec((1, 32)),
            _full_spec((32, 32)), _full_spec((128, 32)), _full_spec((1, 32)),
            _full_spec((32, 32)), _full_spec((1, 32)),
            _full_spec((32, 32)), _full_spec((1, 32)),
            _full_spec((32, 32)), _full_spec((1, 32)),
        ],
        out_specs=[
            _xspec(128), _xspec(32),
            tab_spec, tab_spec, tab_spec, tab_spec,
        ],
        out_shape=[
            jax.ShapeDtypeStruct((L, N, 128), jnp.float32),
            jax.ShapeDtypeStruct((L, N, 32), jnp.float32),
            tab_shape, tab_shape, tab_shape, tab_shape,
        ],
    )(pe_emb, x,
      f['W'][:32], f['W'][32:], f['b'][None],
      f['Wi'][:32], f['Wi'][32:], f['bi'][None],
      f['Wo'][:32], f['Wo'][32:], f['bo'][None],
      pp['W'], pp['b'][None], pp['Wi'], pp['bi'][None],
      pp['Wo'], pp['bo'][None])


def _sc_body(pin_h, pout_h, pepin_h, pepout_h, sg_h, ss_h, dg_h, ds_h,
             zeros_h, inc_h, outg_h, peinc_h, peoutg_h,
             gidx_v, sidx_v, rows0_v, rows1_v, acc_sh,
             gsem0, gsem1, ssem0, ssem1):
    cid = lax.axis_index("c")
    sid = lax.axis_index("s")
    idx_row0 = cid * (IDX_ROWS // 2) + sid * SC_J
    fl0 = sid * 624
    flr = 640
    rows = (rows0_v, rows1_v)
    gsems = (gsem0, gsem1)
    ssems = (ssem0, ssem1)

    def gather(j, b, tab_h):
        pltpu.async_copy(tab_h.at[gidx_v.at[j]], rows[b], gsems[b])

    def scatter(j, b):
        pltpu.async_copy(rows[b], acc_sh.at[sidx_v.at[j]], ssems[b],
                         add=True)

    for tab_h, gh, sh, out_h in (
            (pin_h, sg_h, ds_h, inc_h),
            (pout_h, dg_h, ss_h, outg_h),
            (pepin_h, sg_h, ds_h, peinc_h),
            (pepout_h, dg_h, ss_h, peoutg_h)):
        pltpu.sync_copy(zeros_h.at[pl.ds(fl0, flr)], acc_sh.at[pl.ds(fl0, flr)])
        pltpu.sync_copy(gh.at[pl.ds(idx_row0, SC_J)], gidx_v)
        pltpu.sync_copy(sh.at[pl.ds(idx_row0, SC_J)], sidx_v)
        plsc.subcore_barrier()

        gather(0, 0, tab_h)

        def jbody(i, carry):
            j0 = 2 * i
            j1 = 2 * i + 1
            pltpu.make_async_copy(tab_h.at[gidx_v.at[j0]], rows[0],
                                  gsems[0]).wait()
            scatter(j0, 0)

            @pl.when(i > 0)
            def _():
                pltpu.make_async_copy(rows[1], acc_sh.at[sidx_v.at[j1]],
                                      ssems[1]).wait()

            gather(j1, 1, tab_h)
            pltpu.make_async_copy(tab_h.at[gidx_v.at[j1]], rows[1],
                                  gsems[1]).wait()
            scatter(j1, 1)

            @pl.when(j1 + 1 < SC_J)
            def _():
                pltpu.make_async_copy(rows[0], acc_sh.at[sidx_v.at[j0]],
                                      ssems[0]).wait()
                gather(j1 + 1, 0, tab_h)

            return carry

        lax.fori_loop(0, SC_J // 2, jbody, 0)
        pltpu.make_async_copy(rows[0], acc_sh.at[sidx_v.at[0]],
                              ssems[0]).wait()
        pltpu.make_async_copy(rows[1], acc_sh.at[sidx_v.at[0]],
                              ssems[1]).wait()
        plsc.subcore_barrier()
        pltpu.sync_copy(acc_sh.at[pl.ds(fl0, flr)],
                        out_h.at[cid].at[pl.ds(fl0, flr)])
        plsc.subcore_barrier()


def _call_sc(pin_t, pout_t, pepin_t, pepout_t, sg, ss, dg, ds_, zeros_nd):
    mesh = plsc.VectorSubcoreMesh(core_axis_name="c", subcore_axis_name="s")
    fn = pl.kernel(
        _sc_body,
        out_type=[jax.ShapeDtypeStruct((2, N, 128), jnp.float32)] * 4,
        mesh=mesh,
        scratch_types=[
            pltpu.VMEM((SC_J, SC_C), jnp.int32),
            pltpu.VMEM((SC_J, SC_C), jnp.int32),
            pltpu.VMEM((SC_C, 128), jnp.float32),
            pltpu.VMEM((SC_C, 128), jnp.float32),
            pltpu.VMEM_SHARED((N_ACC, 128), jnp.float32),
            pltpu.SemaphoreType.DMA,
            pltpu.SemaphoreType.DMA,
            pltpu.SemaphoreType.DMA,
            pltpu.SemaphoreType.DMA,
        ],
    )
    return fn(pin_t.reshape(N, 128), pout_t.reshape(N, 128),
              pepin_t.reshape(N, 128), pepout_t.reshape(N, 128),
              sg, ss, dg, ds_, zeros_nd)


def _k2_body(proj_ref, inc_ref, outg_ref, peproj_ref, pei_ref, peo_ref,
             g1_ref, be1_ref, w2_ref, b2_ref, g2_ref, be2_ref,
             pw2_ref, pb2_ref, wk_ref, bk_ref, wv_ref,
             xn_ref, pen_ref, ktv_ref):
    n = pl.program_id(0)
    for l in range(L):
        agg_in = inc_ref[0, :, l, :] + inc_ref[1, :, l, :]
        agg_out = outg_ref[0, :, l, :] + outg_ref[1, :, l, :]
        out = jnp.concatenate([proj_ref[l], agg_in, agg_out], axis=-1)
        h1 = _ln(_silu(out), g1_ref[...], be1_ref[...])
        h2 = h1 @ w2_ref[...] + b2_ref[...]
        xn = _ln(_silu(h2), g2_ref[...], be2_ref[...])
        xn_ref[l] = xn

        pe_in = pei_ref[0, :, l, :] + pei_ref[1, :, l, :]
        pe_out = peo_ref[0, :, l, :] + peo_ref[1, :, l, :]
        pout = jnp.concatenate([peproj_ref[l], pe_in, pe_out], axis=-1)
        pen_ref[l] = jnp.tanh(jnp.tanh(pout) @ pw2_ref[...] + pb2_ref[...])

        k = _elu1(xn @ wk_ref[...] + bk_ref[...])
        v = xn @ wv_ref[...]
        ktv = lax.dot_general(k, v, (((0,), (0,)), ((), ())),
                              preferred_element_type=jnp.float32)

        @pl.when(n == 0)
        def _():
            ktv_ref[l] = ktv

        @pl.when(n != 0)
        def _():
            ktv_ref[l] += ktv


def _call_k2(proj, incP, outgP, peproj, peincP, peoutgP, lp):
    f, pp, ln_ = lp['feat'], lp['pe'], lp['lin']
    part = pl.BlockSpec((2, BN, L, 32), lambda n: (0, n, 0, 0))
    return pl.pallas_call(
        _k2_body,
        grid=(NB,),
        in_specs=[
            _xspec(128), part, part, _xspec(32), part, part,
            _full_spec((1, 192)), _full_spec((1, 192)),
            _full_spec((192, 128)), _full_spec((1, 128)),
            _full_spec((1, 128)), _full_spec((1, 128)),
            _full_spec((96, 32)), _full_spec((1, 32)),
            _full_spec((128, 64)), _full_spec((1, 64)),
            _full_spec((128, 32)),
        ],
        out_specs=[
            _xspec(128), _xspec(32),
            pl.BlockSpec((L, 64, 32), lambda n: (0, 0, 0)),
        ],
        out_shape=[
            jax.ShapeDtypeStruct((L, N, 128), jnp.float32),
            jax.ShapeDtypeStruct((L, N, 32), jnp.float32),
            jax.ShapeDtypeStruct((L, 64, 32), jnp.float32),
        ],
    )(proj, incP.reshape(2, N, L, 32), outgP.reshape(2, N, L, 32),
      peproj, peincP.reshape(2, N, L, 32), peoutgP.reshape(2, N, L, 32),
      f['g1'][None], f['be1'][None], f['W2'], f['b2'][None],
      f['g2'][None], f['be2'][None],
      pp['W2'], pp['b2'][None],
      ln_['Wk'], ln_['bk'][None], ln_['Wv'])


def _k3_body(xn_ref, ktv_ref, wq_ref, bq_ref, wc_x_ref, wc_a_ref, bc_ref,
             x_ref):
    for l in range(L):
        xn = xn_ref[l]
        q = _elu1(xn @ wq_ref[...] + bq_ref[...])
        att = jnp.dot(q, ktv_ref[l], preferred_element_type=jnp.float32)
        x_ref[l] = _silu(xn @ wc_x_ref[...] + att @ wc_a_ref[...]
                         + bc_ref[...])


def _call_k3(x_new, ktv, lp):
    ln_, cb = lp['lin'], lp['comb']
    wc = cb['Wc']
    return pl.pallas_call(
        _k3_body,
        grid=(NB,),
        in_specs=[
            _xspec(128),
            pl.BlockSpec((L, 64, 32), lambda n: (0, 0, 0)),
            _full_spec((128, 64)), _full_spec((1, 64)),
            _full_spec((128, 128)), _full_spec((32, 128)),
            _full_spec((1, 128)),
        ],
        out_specs=_xspec(128),
        out_shape=jax.ShapeDtypeStruct((L, N, 128), jnp.float32),
    )(x_new, ktv, ln_['Wq'], ln_['bq'][None], wc[:128], wc[128:],
      cb['bc'][None])


def kernel(op_code, features, edge_index, lengths, params):
    p = params
    normal = features[..., :127]
    pe_feat = features[..., 127:]
    opc = op_code.astype(jnp.float32)[..., None]

    src, dst = edge_index[0], edge_index[1]
    pad = E_PAD - E
    zpad = jnp.arange(pad, dtype=jnp.int32) % 128
    npad = N + (jnp.arange(pad, dtype=jnp.int32) % 128)
    sg = jnp.concatenate([src, zpad]).reshape(IDX_ROWS, SC_C)
    ss = jnp.concatenate([src, npad]).reshape(IDX_ROWS, SC_C)
    dg = jnp.concatenate([dst, zpad]).reshape(IDX_ROWS, SC_C)
    ds_ = jnp.concatenate([dst, npad]).reshape(IDX_ROWS, SC_C)
    zeros_nd = jnp.zeros((N, 128), jnp.float32)

    x, pe_emb = _call_k0(opc, normal, pe_feat, p)
    for lp in p['layers']:
        proj, peproj, pin_t, pout_t, pepin_t, pepout_t = _call_k1(pe_emb, x, lp)
        incP, outgP, peincP, peoutgP = _call_sc(
            pin_t, pout_t, pepin_t, pepout_t, sg, ss, dg, ds_, zeros_nd)
        x_new, pe_emb, ktv = _call_k2(
            proj, incP, outgP, peproj, peincP, peoutgP, lp)
        x = _call_k3(x_new, ktv, lp)
    return x


# speedup vs baseline: 2.8380x; 1.5312x over previous
import functools

import jax
import jax.numpy as jnp
from jax import lax
from jax.experimental import pallas as pl
from jax.experimental.pallas import tpu as pltpu
from jax.experimental.pallas import tpu_sc as plsc

L = 4
N = 10000
E = 160000
BN = 1000
NB = N // BN

SC_C = 128
SC_J = 40
E_PAD = 32 * SC_J * SC_C
IDX_ROWS = E_PAD // SC_C
N_ACC = N + 128


def _silu(x):
    return x * jax.nn.sigmoid(x)


def _ln(h, g, b):
    m = jnp.mean(h, axis=-1, keepdims=True)
    d = h - m
    v = jnp.mean(d * d, axis=-1, keepdims=True)
    return d * lax.rsqrt(v + 1e-5) * g + b


def _elu1(z):
    return jnp.where(z > 0, z + 1.0, jnp.exp(jnp.minimum(z, 0.0)))


def _full_spec(shape):
    nd = len(shape)
    return pl.BlockSpec(shape, lambda n, _nd=nd: (0,) * _nd)


def _xspec(d):
    return pl.BlockSpec((L, BN, d), lambda n: (0, n, 0))


def _k0_body(opc_ref, normal_ref, pe_ref, emb_ref, win_e_ref, win_n_ref,
             bin_ref, gin_ref, bein_ref, wpe_ref, bpe_ref,
             x_ref, peemb_ref):
    iota = lax.broadcasted_iota(jnp.int32, (BN, 128), 1).astype(jnp.float32)
    for l in range(L):
        opc = opc_ref[l]
        oh = jnp.where(opc == iota, 1.0, 0.0)
        emb = jnp.dot(oh, emb_ref[...], preferred_element_type=jnp.float32)
        pre = (emb @ win_e_ref[...] + normal_ref[l] @ win_n_ref[...]
               + bin_ref[...])
        x_ref[l] = _ln(_silu(pre), gin_ref[...], bein_ref[...])
        peemb_ref[l] = jnp.tanh(pe_ref[l] @ wpe_ref[...] + bpe_ref[...])


def _call_k0(opc, normal, pe_feat, p):
    win = p['W_in']
    return pl.pallas_call(
        _k0_body,
        grid=(NB,),
        in_specs=[
            pl.BlockSpec((L, BN, 1), lambda n: (0, n, 0)),
            pl.BlockSpec((L, BN, 127), lambda n: (0, n, 0)),
            _xspec(16),
            _full_spec((128, 32)),
            _full_spec((32, 128)),
            _full_spec((127, 128)),
            _full_spec((1, 128)),
            _full_spec((1, 128)),
            _full_spec((1, 128)),
            _full_spec((16, 32)),
            _full_spec((1, 32)),
        ],
        out_specs=[_xspec(128), _xspec(32)],
        out_shape=[
            jax.ShapeDtypeStruct((L, N, 128), jnp.float32),
            jax.ShapeDtypeStruct((L, N, 32), jnp.float32),
        ],
    )(opc, normal, pe_feat, p['emb'], win[:32], win[32:],
      p['b_in'][None], p['g_in'][None], p['be_in'][None],
      p['W_pe'], p['b_pe'][None])


def _k1_body(pe_ref, x_ref, w_pe_ref, w_x_ref, b_ref, wi_pe_ref, wi_x_ref,
             bi_ref, wo_pe_ref, wo_x_ref, bo_ref,
             pw_ref, pb_ref, pwi_ref, pbi_ref, pwo_ref, pbo_ref,
             proj_ref, peproj_ref, pin_ref, pout_ref, pepin_ref, pepout_ref):
    pin, pout, pepin, pepout = [], [], [], []
    for l in range(L):
        pe = pe_ref[l]
        x = x_ref[l]
        proj_ref[l] = pe @ w_pe_ref[...] + x @ w_x_ref[...] + b_ref[...]
        pin.append(pe @ wi_pe_ref[...] + x @ wi_x_ref[...] + bi_ref[...])
        pout.append(pe @ wo_pe_ref[...] + x @ wo_x_ref[...] + bo_ref[...])
        peproj_ref[l] = pe @ pw_ref[...] + pb_ref[...]
        pepin.append(pe @ pwi_ref[...] + pbi_ref[...])
        pepout.append(pe @ pwo_ref[...] + pbo_ref[...])
    pin_ref[...] = jnp.concatenate(pin, axis=-1)
    pout_ref[...] = jnp.concatenate(pout, axis=-1)
    pepin_ref[...] = jnp.concatenate(pepin, axis=-1)
    pepout_ref[...] = jnp.concatenate(pepout, axis=-1)


def _call_k1(pe_emb, x, lp):
    f, pp = lp['feat'], lp['pe']
    tab_spec = pl.BlockSpec((BN, 128), lambda n: (n, 0))
    tab_shape = jax.ShapeDtypeStruct((N, 128), jnp.float32)
    return pl.pallas_call(
        _k1_body,
        grid=(NB,),
        in_specs=[
            _xspec(32),
            _xspec(128),
            _full_spec((32, 128)), _full_spec((128, 128)), _full_spec((1, 128)),
            _full_spec((32, 32)), _full_spec((128, 32)), _full_spec((1, 32)),
            _full_spec((32, 32)), _full_spec((128, 32)), _full_spec((1, 32)),
            _full_spec((32, 32)), _full_spec((1, 32)),
            _full_spec((32, 32)), _full_spec((1, 32)),
            _full_spec((32, 32)), _full_spec((1, 32)),
        ],
        out_specs=[
            _xspec(128), _xspec(32),
            tab_spec, tab_spec, tab_spec, tab_spec,
        ],
        out_shape=[
            jax.ShapeDtypeStruct((L, N, 128), jnp.float32),
            jax.ShapeDtypeStruct((L, N, 32), jnp.float32),
            tab_shape, tab_shape, tab_shape, tab_shape,
        ],
    )(pe_emb, x,
      f['W'][:32], f['W'][32:], f['b'][None],
      f['Wi'][:32], f['Wi'][32:], f['bi'][None],
      f['Wo'][:32], f['Wo'][32:], f['bo'][None],
      pp['W'], pp['b'][None], pp['Wi'], pp['bi'][None],
      pp['Wo'], pp['bo'][None])


def _sc_body(pin_h, pout_h, pepin_h, pepout_h, sg_h, ss_h, dg_h, ds_h,
             zeros_h, inc_h, outg_h, peinc_h, peoutg_h,
             gidx_v, sidx_v, rows0_v, rows1_v, acc_sh,
             gsem0, gsem1, ssem0, ssem1):
    cid = lax.axis_index("c")
    sid = lax.axis_index("s")
    idx_row0 = cid * (IDX_ROWS // 2) + sid * SC_J
    fl0 = sid * 624
    flr = 640
    rows = (rows0_v, rows1_v)
    gsems = (gsem0, gsem1)
    ssems = (ssem0, ssem1)

    def gather(j, b, tab_h):
        pltpu.async_copy(tab_h.at[gidx_v.at[j]], rows[b], gsems[b])

    def scatter(j, b):
        pltpu.async_copy(rows[b], acc_sh.at[sidx_v.at[j]], ssems[b],
                         add=True)

    for tab_h, gh, sh, out_h in (
            (pin_h, sg_h, ds_h, inc_h),
            (pout_h, dg_h, ss_h, outg_h),
            (pepin_h, sg_h, ds_h, peinc_h),
            (pepout_h, dg_h, ss_h, peoutg_h)):
        pltpu.sync_copy(zeros_h.at[pl.ds(fl0, flr)], acc_sh.at[pl.ds(fl0, flr)])
        pltpu.sync_copy(gh.at[pl.ds(idx_row0, SC_J)], gidx_v)
        pltpu.sync_copy(sh.at[pl.ds(idx_row0, SC_J)], sidx_v)
        plsc.subcore_barrier()

        gather(0, 0, tab_h)

        def jbody(i, carry):
            j0 = 2 * i
            j1 = 2 * i + 1
            pltpu.make_async_copy(tab_h.at[gidx_v.at[j0]], rows[0],
                                  gsems[0]).wait()
            scatter(j0, 0)

            @pl.when(i > 0)
            def _():
                pltpu.make_async_copy(rows[1], acc_sh.at[sidx_v.at[j1]],
                                      ssems[1]).wait()

            gather(j1, 1, tab_h)
            pltpu.make_async_copy(tab_h.at[gidx_v.at[j1]], rows[1],
                                  gsems[1]).wait()
            scatter(j1, 1)

            @pl.when(j1 + 1 < SC_J)
            def _():
                pltpu.make_async_copy(rows[0], acc_sh.at[sidx_v.at[j0]],
                                      ssems[0]).wait()
                gather(j1 + 1, 0, tab_h)

            return carry

        lax.fori_loop(0, SC_J // 2, jbody, 0)
        pltpu.make_async_copy(rows[0], acc_sh.at[sidx_v.at[0]],
                              ssems[0]).wait()
        pltpu.make_async_copy(rows[1], acc_sh.at[sidx_v.at[0]],
                              ssems[1]).wait()
        plsc.subcore_barrier()
        pltpu.sync_copy(acc_sh.at[pl.ds(fl0, flr)],
                        out_h.at[cid].at[pl.ds(fl0, flr)])
        plsc.subcore_barrier()


def _call_sc(pin_t, pout_t, pepin_t, pepout_t, sg, ss, dg, ds_, zeros_nd):
    mesh = plsc.VectorSubcoreMesh(core_axis_name="c", subcore_axis_name="s")
    fn = pl.kernel(
        _sc_body,
        out_type=[jax.ShapeDtypeStruct((2, N, 128), jnp.float32)] * 4,
        mesh=mesh,
        scratch_types=[
            pltpu.VMEM((SC_J, SC_C), jnp.int32),
            pltpu.VMEM((SC_J, SC_C), jnp.int32),
            pltpu.VMEM((SC_C, 128), jnp.float32),
            pltpu.VMEM((SC_C, 128), jnp.float32),
            pltpu.VMEM_SHARED((N_ACC, 128), jnp.float32),
            pltpu.SemaphoreType.DMA,
            pltpu.SemaphoreType.DMA,
            pltpu.SemaphoreType.DMA,
            pltpu.SemaphoreType.DMA,
        ],
    )
    return fn(pin_t, pout_t, pepin_t, pepout_t, sg, ss, dg, ds_, zeros_nd)


def _k2_body(proj_ref, inc_ref, outg_ref, peproj_ref, pei_ref, peo_ref,
             g1_ref, be1_ref, w2_ref, b2_ref, g2_ref, be2_ref,
             pw2_ref, pb2_ref, wk_ref, bk_ref, wv_ref,
             xn_ref, pen_ref, ktv_ref):
    n = pl.program_id(0)
    inc = inc_ref[0] + inc_ref[1]
    outg = outg_ref[0] + outg_ref[1]
    pei = pei_ref[0] + pei_ref[1]
    peo = peo_ref[0] + peo_ref[1]
    for l in range(L):
        agg_in = inc[:, l * 32:(l + 1) * 32]
        agg_out = outg[:, l * 32:(l + 1) * 32]
        out = jnp.concatenate([proj_ref[l], agg_in, agg_out], axis=-1)
        h1 = _ln(_silu(out), g1_ref[...], be1_ref[...])
        h2 = h1 @ w2_ref[...] + b2_ref[...]
        xn = _ln(_silu(h2), g2_ref[...], be2_ref[...])
        xn_ref[l] = xn

        pe_in = pei[:, l * 32:(l + 1) * 32]
        pe_out = peo[:, l * 32:(l + 1) * 32]
        pout = jnp.concatenate([peproj_ref[l], pe_in, pe_out], axis=-1)
        pen_ref[l] = jnp.tanh(jnp.tanh(pout) @ pw2_ref[...] + pb2_ref[...])

        k = _elu1(xn @ wk_ref[...] + bk_ref[...])
        v = xn @ wv_ref[...]
        ktv = lax.dot_general(k, v, (((0,), (0,)), ((), ())),
                              preferred_element_type=jnp.float32)

        @pl.when(n == 0)
        def _():
            ktv_ref[l] = ktv

        @pl.when(n != 0)
        def _():
            ktv_ref[l] += ktv


def _call_k2(proj, incP, outgP, peproj, peincP, peoutgP, lp):
    f, pp, ln_ = lp['feat'], lp['pe'], lp['lin']
    part = pl.BlockSpec((2, BN, 128), lambda n: (0, n, 0))
    return pl.pallas_call(
        _k2_body,
        grid=(NB,),
        in_specs=[
            _xspec(128), part, part, _xspec(32), part, part,
            _full_spec((1, 192)), _full_spec((1, 192)),
            _full_spec((192, 128)), _full_spec((1, 128)),
            _full_spec((1, 128)), _full_spec((1, 128)),
            _full_spec((96, 32)), _full_spec((1, 32)),
            _full_spec((128, 64)), _full_spec((1, 64)),
            _full_spec((128, 32)),
        ],
        out_specs=[
            _xspec(128), _xspec(32),
            pl.BlockSpec((L, 64, 32), lambda n: (0, 0, 0)),
        ],
        out_shape=[
            jax.ShapeDtypeStruct((L, N, 128), jnp.float32),
            jax.ShapeDtypeStruct((L, N, 32), jnp.float32),
            jax.ShapeDtypeStruct((L, 64, 32), jnp.float32),
        ],
    )(proj, incP, outgP, peproj, peincP, peoutgP,
      f['g1'][None], f['be1'][None], f['W2'], f['b2'][None],
      f['g2'][None], f['be2'][None],
      pp['W2'], pp['b2'][None],
      ln_['Wk'], ln_['bk'][None], ln_['Wv'])


def _k3_body(xn_ref, ktv_ref, wq_ref, bq_ref, wc_x_ref, wc_a_ref, bc_ref,
             x_ref):
    for l in range(L):
        xn = xn_ref[l]
        q = _elu1(xn @ wq_ref[...] + bq_ref[...])
        att = jnp.dot(q, ktv_ref[l], preferred_element_type=jnp.float32)
        x_ref[l] = _silu(xn @ wc_x_ref[...] + att @ wc_a_ref[...]
                         + bc_ref[...])


def _call_k3(x_new, ktv, lp):
    ln_, cb = lp['lin'], lp['comb']
    wc = cb['Wc']
    return pl.pallas_call(
        _k3_body,
        grid=(NB,),
        in_specs=[
            _xspec(128),
            pl.BlockSpec((L, 64, 32), lambda n: (0, 0, 0)),
            _full_spec((128, 64)), _full_spec((1, 64)),
            _full_spec((128, 128)), _full_spec((32, 128)),
            _full_spec((1, 128)),
        ],
        out_specs=_xspec(128),
        out_shape=jax.ShapeDtypeStruct((L, N, 128), jnp.float32),
    )(x_new, ktv, ln_['Wq'], ln_['bq'][None], wc[:128], wc[128:],
      cb['bc'][None])


def kernel(op_code, features, edge_index, lengths, params):
    p = params
    normal = features[..., :127]
    pe_feat = features[..., 127:]
    opc = op_code.astype(jnp.float32)[..., None]

    src, dst = edge_index[0], edge_index[1]
    pad = E_PAD - E
    zpad = jnp.arange(pad, dtype=jnp.int32) % 128
    npad = N + (jnp.arange(pad, dtype=jnp.int32) % 128)
    sg = jnp.concatenate([src, zpad]).reshape(IDX_ROWS, SC_C)
    ss = jnp.concatenate([src, npad]).reshape(IDX_ROWS, SC_C)
    dg = jnp.concatenate([dst, zpad]).reshape(IDX_ROWS, SC_C)
    ds_ = jnp.concatenate([dst, npad]).reshape(IDX_ROWS, SC_C)
    zeros_nd = jnp.zeros((N, 128), jnp.float32)

    x, pe_emb = _call_k0(opc, normal, pe_feat, p)
    for lp in p['layers']:
        proj, peproj, pin_t, pout_t, pepin_t, pepout_t = _call_k1(pe_emb, x, lp)
        incP, outgP, peincP, peoutgP = _call_sc(
            pin_t, pout_t, pepin_t, pepout_t, sg, ss, dg, ds_, zeros_nd)
        x_new, pe_emb, ktv = _call_k2(
            proj, incP, outgP, peproj, peincP, peoutgP, lp)
        x = _call_k3(x_new, ktv, lp)
    return x


# speedup vs baseline: 3.2608x; 1.1490x over previous
import functools

import jax
import jax.numpy as jnp
from jax import lax
from jax.experimental import pallas as pl
from jax.experimental.pallas import tpu as pltpu
from jax.experimental.pallas import tpu_sc as plsc

L = 4
N = 10000
E = 160000
BN = 1000
NB = N // BN

SC_C = 128
SC_J = 40
E_PAD = 32 * SC_J * SC_C
IDX_ROWS = E_PAD // SC_C
N_ACC = N + 128


def _silu(x):
    return x * jax.nn.sigmoid(x)


def _ln(h, g, b):
    m = jnp.mean(h, axis=-1, keepdims=True)
    d = h - m
    v = jnp.mean(d * d, axis=-1, keepdims=True)
    return d * lax.rsqrt(v + 1e-5) * g + b


def _elu1(z):
    return jnp.where(z > 0, z + 1.0, jnp.exp(jnp.minimum(z, 0.0)))


def _full_spec(shape):
    nd = len(shape)
    return pl.BlockSpec(shape, lambda n, _nd=nd: (0,) * _nd)


def _xspec(d):
    return pl.BlockSpec((L, BN, d), lambda n: (0, n, 0))


def _k0_body(opc_ref, normal_ref, pe_ref, emb_ref, win_e_ref, win_n_ref,
             bin_ref, gin_ref, bein_ref, wpe_ref, bpe_ref,
             x_ref, peemb_ref):
    iota = lax.broadcasted_iota(jnp.int32, (BN, 128), 1).astype(jnp.float32)
    for l in range(L):
        opc = opc_ref[l]
        oh = jnp.where(opc == iota, 1.0, 0.0)
        emb = jnp.dot(oh, emb_ref[...], preferred_element_type=jnp.float32)
        pre = (emb @ win_e_ref[...] + normal_ref[l] @ win_n_ref[...]
               + bin_ref[...])
        x_ref[l] = _ln(_silu(pre), gin_ref[...], bein_ref[...])
        peemb_ref[l] = jnp.tanh(pe_ref[l] @ wpe_ref[...] + bpe_ref[...])


def _call_k0(opc, normal, pe_feat, p):
    win = p['W_in']
    return pl.pallas_call(
        _k0_body,
        grid=(NB,),
        in_specs=[
            pl.BlockSpec((L, BN, 1), lambda n: (0, n, 0)),
            pl.BlockSpec((L, BN, 127), lambda n: (0, n, 0)),
            _xspec(16),
            _full_spec((128, 32)),
            _full_spec((32, 128)),
            _full_spec((127, 128)),
            _full_spec((1, 128)),
            _full_spec((1, 128)),
            _full_spec((1, 128)),
            _full_spec((16, 32)),
            _full_spec((1, 32)),
        ],
        out_specs=[_xspec(128), _xspec(32)],
        out_shape=[
            jax.ShapeDtypeStruct((L, N, 128), jnp.float32),
            jax.ShapeDtypeStruct((L, N, 32), jnp.float32),
        ],
    )(opc, normal, pe_feat, p['emb'], win[:32], win[32:],
      p['b_in'][None], p['g_in'][None], p['be_in'][None],
      p['W_pe'], p['b_pe'][None])


def _k1_body(pe_ref, x_ref, w_pe_ref, w_x_ref, b_ref, wi_pe_ref, wi_x_ref,
             bi_ref, wo_pe_ref, wo_x_ref, bo_ref,
             pw_ref, pb_ref, pwi_ref, pbi_ref, pwo_ref, pbo_ref,
             proj_ref, peproj_ref, pin_ref, pout_ref, pepin_ref, pepout_ref):
    pin, pout, pepin, pepout = [], [], [], []
    for l in range(L):
        pe = pe_ref[l]
        x = x_ref[l]
        proj_ref[l] = pe @ w_pe_ref[...] + x @ w_x_ref[...] + b_ref[...]
        pin.append(pe @ wi_pe_ref[...] + x @ wi_x_ref[...] + bi_ref[...])
        pout.append(pe @ wo_pe_ref[...] + x @ wo_x_ref[...] + bo_ref[...])
        peproj_ref[l] = pe @ pw_ref[...] + pb_ref[...]
        pepin.append(pe @ pwi_ref[...] + pbi_ref[...])
        pepout.append(pe @ pwo_ref[...] + pbo_ref[...])
    pin_ref[...] = jnp.concatenate(pin, axis=-1)
    pout_ref[...] = jnp.concatenate(pout, axis=-1)
    pepin_ref[...] = jnp.concatenate(pepin, axis=-1)
    pepout_ref[...] = jnp.concatenate(pepout, axis=-1)


def _call_k1(pe_emb, x, lp):
    f, pp = lp['feat'], lp['pe']
    tab_spec = pl.BlockSpec((BN, 128), lambda n: (n, 0))
    tab_shape = jax.ShapeDtypeStruct((N, 128), jnp.float32)
    return pl.pallas_call(
        _k1_body,
        grid=(NB,),
        in_specs=[
            _xspec(32),
            _xspec(128),
            _full_spec((32, 128)), _full_spec((128, 128)), _full_spec((1, 128)),
            _full_spec((32, 32)), _full_spec((128, 32)), _full_spec((1, 32)),
            _full_spec((32, 32)), _full_spec((128, 32)), _full_spec((1, 32)),
            _full_spec((32, 32)), _full_spec((1, 32)),
            _full_spec((32, 32)), _full_spec((1, 32)),
            _full_spec((32, 32)), _full_spec((1, 32)),
        ],
        out_specs=[
            _xspec(128), _xspec(32),
            tab_spec, tab_spec, tab_spec, tab_spec,
        ],
        out_shape=[
            jax.ShapeDtypeStruct((L, N, 128), jnp.float32),
            jax.ShapeDtypeStruct((L, N, 32), jnp.float32),
            tab_shape, tab_shape, tab_shape, tab_shape,
        ],
    )(pe_emb, x,
      f['W'][:32], f['W'][32:], f['b'][None],
      f['Wi'][:32], f['Wi'][32:], f['bi'][None],
      f['Wo'][:32], f['Wo'][32:], f['bo'][None],
      pp['W'], pp['b'][None], pp['Wi'], pp['bi'][None],
      pp['Wo'], pp['bo'][None])


def _sc_body(pin_h, pout_h, pepin_h, pepout_h, sg_h, ss_h, dg_h, ds_h,
             zeros_h, inc_h, outg_h, peinc_h, peoutg_h,
             gidx_v, sidx_v, rows0_v, rows1_v, acc_sh,
             gsem0, gsem1, ssem0, ssem1):
    cid = lax.axis_index("c")
    sid = lax.axis_index("s")
    idx_row0 = cid * (IDX_ROWS // 2) + sid * SC_J
    fl0 = sid * 624
    flr = 640
    rows = (rows0_v, rows1_v)
    gsems = (gsem0, gsem1)
    ssems = (ssem0, ssem1)

    def gather(j, b, tab_h):
        pltpu.async_copy(tab_h.at[gidx_v.at[j]], rows[b], gsems[b])

    def scatter(j, b):
        pltpu.async_copy(rows[b], acc_sh.at[sidx_v.at[j]], ssems[b],
                         add=True)

    for tab_h, gh, sh, out_h in (
            (pin_h, sg_h, ds_h, inc_h),
            (pout_h, dg_h, ss_h, outg_h),
            (pepin_h, sg_h, ds_h, peinc_h),
            (pepout_h, dg_h, ss_h, peoutg_h)):
        pltpu.sync_copy(zeros_h.at[pl.ds(fl0, flr)], acc_sh.at[pl.ds(fl0, flr)])
        pltpu.sync_copy(gh.at[pl.ds(idx_row0, SC_J)], gidx_v)
        pltpu.sync_copy(sh.at[pl.ds(idx_row0, SC_J)], sidx_v)
        plsc.subcore_barrier()

        gather(0, 0, tab_h)
        gather(1, 1, tab_h)

        def jbody(i, carry):
            j0 = 2 * i
            j1 = 2 * i + 1
            pltpu.make_async_copy(tab_h.at[gidx_v.at[j0]], rows[0],
                                  gsems[0]).wait()

            @pl.when(j1 + 1 < SC_J)
            def _():
                gather(j1 + 1, 0, tab_h)

            pltpu.make_async_copy(tab_h.at[gidx_v.at[j1]], rows[1],
                                  gsems[1]).wait()

            @pl.when(j1 + 2 < SC_J)
            def _():
                gather(j1 + 2, 1, tab_h)

            return carry

        lax.fori_loop(0, SC_J // 2, jbody, 0)
        plsc.subcore_barrier()
        pltpu.sync_copy(acc_sh.at[pl.ds(fl0, flr)],
                        out_h.at[cid].at[pl.ds(fl0, flr)])
        plsc.subcore_barrier()


def _call_sc(pin_t, pout_t, pepin_t, pepout_t, sg, ss, dg, ds_, zeros_nd):
    mesh = plsc.VectorSubcoreMesh(core_axis_name="c", subcore_axis_name="s")
    fn = pl.kernel(
        _sc_body,
        out_type=[jax.ShapeDtypeStruct((2, N, 128), jnp.float32)] * 4,
        mesh=mesh,
        scratch_types=[
            pltpu.VMEM((SC_J, SC_C), jnp.int32),
            pltpu.VMEM((SC_J, SC_C), jnp.int32),
            pltpu.VMEM((SC_C, 128), jnp.float32),
            pltpu.VMEM((SC_C, 128), jnp.float32),
            pltpu.VMEM_SHARED((N_ACC, 128), jnp.float32),
            pltpu.SemaphoreType.DMA,
            pltpu.SemaphoreType.DMA,
            pltpu.SemaphoreType.DMA,
            pltpu.SemaphoreType.DMA,
        ],
    )
    return fn(pin_t, pout_t, pepin_t, pepout_t, sg, ss, dg, ds_, zeros_nd)


def _k2_body(proj_ref, inc_ref, outg_ref, peproj_ref, pei_ref, peo_ref,
             g1_ref, be1_ref, w2_ref, b2_ref, g2_ref, be2_ref,
             pw2_ref, pb2_ref, wk_ref, bk_ref, wv_ref,
             xn_ref, pen_ref, ktv_ref):
    n = pl.program_id(0)
    inc = inc_ref[0] + inc_ref[1]
    outg = outg_ref[0] + outg_ref[1]
    pei = pei_ref[0] + pei_ref[1]
    peo = peo_ref[0] + peo_ref[1]
    for l in range(L):
        agg_in = inc[:, l * 32:(l + 1) * 32]
        agg_out = outg[:, l * 32:(l + 1) * 32]
        out = jnp.concatenate([proj_ref[l], agg_in, agg_out], axis=-1)
        h1 = _ln(_silu(out), g1_ref[...], be1_ref[...])
        h2 = h1 @ w2_ref[...] + b2_ref[...]
        xn = _ln(_silu(h2), g2_ref[...], be2_ref[...])
        xn_ref[l] = xn

        pe_in = pei[:, l * 32:(l + 1) * 32]
        pe_out = peo[:, l * 32:(l + 1) * 32]
        pout = jnp.concatenate([peproj_ref[l], pe_in, pe_out], axis=-1)
        pen_ref[l] = jnp.tanh(jnp.tanh(pout) @ pw2_ref[...] + pb2_ref[...])

        k = _elu1(xn @ wk_ref[...] + bk_ref[...])
        v = xn @ wv_ref[...]
        ktv = lax.dot_general(k, v, (((0,), (0,)), ((), ())),
                              preferred_element_type=jnp.float32)

        @pl.when(n == 0)
        def _():
            ktv_ref[l] = ktv

        @pl.when(n != 0)
        def _():
            ktv_ref[l] += ktv


def _call_k2(proj, incP, outgP, peproj, peincP, peoutgP, lp):
    f, pp, ln_ = lp['feat'], lp['pe'], lp['lin']
    part = pl.BlockSpec((2, BN, 128), lambda n: (0, n, 0))
    return pl.pallas_call(
        _k2_body,
        grid=(NB,),
        in_specs=[
            _xspec(128), part, part, _xspec(32), part, part,
            _full_spec((1, 192)), _full_spec((1, 192)),
            _full_spec((192, 128)), _full_spec((1, 128)),
            _full_spec((1, 128)), _full_spec((1, 128)),
            _full_spec((96, 32)), _full_spec((1, 32)),
            _full_spec((128, 64)), _full_spec((1, 64)),
            _full_spec((128, 32)),
        ],
        out_specs=[
            _xspec(128), _xspec(32),
            pl.BlockSpec((L, 64, 32), lambda n: (0, 0, 0)),
        ],
        out_shape=[
            jax.ShapeDtypeStruct((L, N, 128), jnp.float32),
            jax.ShapeDtypeStruct((L, N, 32), jnp.float32),
            jax.ShapeDtypeStruct((L, 64, 32), jnp.float32),
        ],
    )(proj, incP, outgP, peproj, peincP, peoutgP,
      f['g1'][None], f['be1'][None], f['W2'], f['b2'][None],
      f['g2'][None], f['be2'][None],
      pp['W2'], pp['b2'][None],
      ln_['Wk'], ln_['bk'][None], ln_['Wv'])


def _k3_body(xn_ref, ktv_ref, wq_ref, bq_ref, wc_x_ref, wc_a_ref, bc_ref,
             x_ref):
    for l in range(L):
        xn = xn_ref[l]
        q = _elu1(xn @ wq_ref[...] + bq_ref[...])
        att = jnp.dot(q, ktv_ref[l], preferred_element_type=jnp.float32)
        x_ref[l] = _silu(xn @ wc_x_ref[...] + att @ wc_a_ref[...]
                         + bc_ref[...])


def _call_k3(x_new, ktv, lp):
    ln_, cb = lp['lin'], lp['comb']
    wc = cb['Wc']
    return pl.pallas_call(
        _k3_body,
        grid=(NB,),
        in_specs=[
            _xspec(128),
            pl.BlockSpec((L, 64, 32), lambda n: (0, 0, 0)),
            _full_spec((128, 64)), _full_spec((1, 64)),
            _full_spec((128, 128)), _full_spec((32, 128)),
            _full_spec((1, 128)),
        ],
        out_specs=_xspec(128),
        out_shape=jax.ShapeDtypeStruct((L, N, 128), jnp.float32),
    )(x_new, ktv, ln_['Wq'], ln_['bq'][None], wc[:128], wc[128:],
      cb['bc'][None])


def kernel(op_code, features, edge_index, lengths, params):
    p = params
    normal = features[..., :127]
    pe_feat = features[..., 127:]
    opc = op_code.astype(jnp.float32)[..., None]

    src, dst = edge_index[0], edge_index[1]
    pad = E_PAD - E
    zpad = jnp.arange(pad, dtype=jnp.int32) % 128
    npad = N + (jnp.arange(pad, dtype=jnp.int32) % 128)
    sg = jnp.concatenate([src, zpad]).reshape(IDX_ROWS, SC_C)
    ss = jnp.concatenate([src, npad]).reshape(IDX_ROWS, SC_C)
    dg = jnp.concatenate([dst, zpad]).reshape(IDX_ROWS, SC_C)
    ds_ = jnp.concatenate([dst, npad]).reshape(IDX_ROWS, SC_C)
    zeros_nd = jnp.zeros((N, 128), jnp.float32)

    x, pe_emb = _call_k0(opc, normal, pe_feat, p)
    for lp in p['layers']:
        proj, peproj, pin_t, pout_t, pepin_t, pepout_t = _call_k1(pe_emb, x, lp)
        incP, outgP, peincP, peoutgP = _call_sc(
            pin_t, pout_t, pepin_t, pepout_t, sg, ss, dg, ds_, zeros_nd)
        x_new, pe_emb, ktv = _call_k2(
            proj, incP, outgP, peproj, peincP, peoutgP, lp)
        x = _call_k3(x_new, ktv, lp)
    return x


# speedup vs baseline: 3.3396x; 1.0242x over previous
import functools

import jax
import jax.numpy as jnp
from jax import lax
from jax.experimental import pallas as pl
from jax.experimental.pallas import tpu as pltpu
from jax.experimental.pallas import tpu_sc as plsc

L = 4
N = 10000
E = 160000
BN = 1000
NB = N // BN

SC_C = 128
SC_J = 40
E_PAD = 32 * SC_J * SC_C
IDX_ROWS = E_PAD // SC_C
N_ACC = N + 128


def _silu(x):
    return x * jax.nn.sigmoid(x)


def _ln(h, g, b):
    m = jnp.mean(h, axis=-1, keepdims=True)
    d = h - m
    v = jnp.mean(d * d, axis=-1, keepdims=True)
    return d * lax.rsqrt(v + 1e-5) * g + b


def _elu1(z):
    return jnp.where(z > 0, z + 1.0, jnp.exp(jnp.minimum(z, 0.0)))


def _full_spec(shape):
    nd = len(shape)
    return pl.BlockSpec(shape, lambda n, _nd=nd: (0,) * _nd)


def _xspec(d):
    return pl.BlockSpec((L, BN, d), lambda n: (0, n, 0))


def _k0_body(opc_ref, normal_ref, pe_ref, emb_ref, win_e_ref, win_n_ref,
             bin_ref, gin_ref, bein_ref, wpe_ref, bpe_ref,
             x_ref, peemb_ref):
    iota = lax.broadcasted_iota(jnp.int32, (BN, 128), 1).astype(jnp.float32)
    for l in range(L):
        opc = opc_ref[l]
        oh = jnp.where(opc == iota, 1.0, 0.0)
        emb = jnp.dot(oh, emb_ref[...], preferred_element_type=jnp.float32)
        pre = (emb @ win_e_ref[...] + normal_ref[l] @ win_n_ref[...]
               + bin_ref[...])
        x_ref[l] = _ln(_silu(pre), gin_ref[...], bein_ref[...])
        peemb_ref[l] = jnp.tanh(pe_ref[l] @ wpe_ref[...] + bpe_ref[...])


def _call_k0(opc, normal, pe_feat, p):
    win = p['W_in']
    return pl.pallas_call(
        _k0_body,
        grid=(NB,),
        in_specs=[
            pl.BlockSpec((L, BN, 1), lambda n: (0, n, 0)),
            pl.BlockSpec((L, BN, 127), lambda n: (0, n, 0)),
            _xspec(16),
            _full_spec((128, 32)),
            _full_spec((32, 128)),
            _full_spec((127, 128)),
            _full_spec((1, 128)),
            _full_spec((1, 128)),
            _full_spec((1, 128)),
            _full_spec((16, 32)),
            _full_spec((1, 32)),
        ],
        out_specs=[_xspec(128), _xspec(32)],
        out_shape=[
            jax.ShapeDtypeStruct((L, N, 128), jnp.float32),
            jax.ShapeDtypeStruct((L, N, 32), jnp.float32),
        ],
    )(opc, normal, pe_feat, p['emb'], win[:32], win[32:],
      p['b_in'][None], p['g_in'][None], p['be_in'][None],
      p['W_pe'], p['b_pe'][None])


def _k1_body(pe_ref, x_ref, w_pe_ref, w_x_ref, b_ref, wi_pe_ref, wi_x_ref,
             bi_ref, wo_pe_ref, wo_x_ref, bo_ref,
             pw_ref, pb_ref, pwi_ref, pbi_ref, pwo_ref, pbo_ref,
             proj_ref, peproj_ref, pin_ref, pout_ref, pepin_ref, pepout_ref):
    pin, pout, pepin, pepout = [], [], [], []
    for l in range(L):
        pe = pe_ref[l]
        x = x_ref[l]
        proj_ref[l] = pe @ w_pe_ref[...] + x @ w_x_ref[...] + b_ref[...]
        pin.append(pe @ wi_pe_ref[...] + x @ wi_x_ref[...] + bi_ref[...])
        pout.append(pe @ wo_pe_ref[...] + x @ wo_x_ref[...] + bo_ref[...])
        peproj_ref[l] = pe @ pw_ref[...] + pb_ref[...]
        pepin.append(pe @ pwi_ref[...] + pbi_ref[...])
        pepout.append(pe @ pwo_ref[...] + pbo_ref[...])
    pin_ref[...] = jnp.concatenate(pin, axis=-1)
    pout_ref[...] = jnp.concatenate(pout, axis=-1)
    pepin_ref[...] = jnp.concatenate(pepin, axis=-1)
    pepout_ref[...] = jnp.concatenate(pepout, axis=-1)


def _call_k1(pe_emb, x, lp):
    f, pp = lp['feat'], lp['pe']
    tab_spec = pl.BlockSpec((BN, 128), lambda n: (n, 0))
    tab_shape = jax.ShapeDtypeStruct((N, 128), jnp.float32)
    return pl.pallas_call(
        _k1_body,
        grid=(NB,),
        in_specs=[
            _xspec(32),
            _xspec(128),
            _full_spec((32, 128)), _full_spec((128, 128)), _full_spec((1, 128)),
            _full_spec((32, 32)), _full_spec((128, 32)), _full_spec((1, 32)),
            _full_spec((32, 32)), _full_spec((128, 32)), _full_spec((1, 32)),
            _full_spec((32, 32)), _full_spec((1, 32)),
            _full_spec((32, 32)), _full_spec((1, 32)),
            _full_spec((32, 32)), _full_spec((1, 32)),
        ],
        out_specs=[
            _xspec(128), _xspec(32),
            tab_spec, tab_spec, tab_spec, tab_spec,
        ],
        out_shape=[
            jax.ShapeDtypeStruct((L, N, 128), jnp.float32),
            jax.ShapeDtypeStruct((L, N, 32), jnp.float32),
            tab_shape, tab_shape, tab_shape, tab_shape,
        ],
    )(pe_emb, x,
      f['W'][:32], f['W'][32:], f['b'][None],
      f['Wi'][:32], f['Wi'][32:], f['bi'][None],
      f['Wo'][:32], f['Wo'][32:], f['bo'][None],
      pp['W'], pp['b'][None], pp['Wi'], pp['bi'][None],
      pp['Wo'], pp['bo'][None])


def _sc_body(tin_h, tout_h, sg_h, ss_h, dg_h, ds_h,
             zeros_h, inc_h, outg_h,
             gidx_v, sidx_v, rows0_v, rows1_v, acc_sh,
             gsem0, gsem1, ssem0, ssem1):
    cid = lax.axis_index("c")
    sid = lax.axis_index("s")
    idx_row0 = cid * (IDX_ROWS // 2) + sid * SC_J
    fl0 = sid * 624
    flr = 640
    rows = (rows0_v, rows1_v)
    gsems = (gsem0, gsem1)
    ssems = (ssem0, ssem1)

    def gather(j, b, tab_h):
        pltpu.async_copy(tab_h.at[gidx_v.at[j]], rows[b], gsems[b])

    def scatter(j, b):
        pltpu.async_copy(rows[b], acc_sh.at[sidx_v.at[j]], ssems[b],
                         add=True)

    for tab_h, gh, sh, out_h in (
            (tin_h, sg_h, ds_h, inc_h),
            (tout_h, dg_h, ss_h, outg_h)):
        pltpu.sync_copy(zeros_h.at[pl.ds(fl0, flr)], acc_sh.at[pl.ds(fl0, flr)])
        pltpu.sync_copy(gh.at[pl.ds(idx_row0, SC_J)], gidx_v)
        pltpu.sync_copy(sh.at[pl.ds(idx_row0, SC_J)], sidx_v)
        plsc.subcore_barrier()

        gather(0, 0, tab_h)

        def jbody(i, carry):
            j0 = 2 * i
            j1 = 2 * i + 1
            pltpu.make_async_copy(tab_h.at[gidx_v.at[j0]], rows[0],
                                  gsems[0]).wait()
            scatter(j0, 0)

            @pl.when(i > 0)
            def _():
                pltpu.make_async_copy(rows[1], acc_sh.at[sidx_v.at[j1]],
                                      ssems[1]).wait()

            gather(j1, 1, tab_h)
            pltpu.make_async_copy(tab_h.at[gidx_v.at[j1]], rows[1],
                                  gsems[1]).wait()
            scatter(j1, 1)

            @pl.when(j1 + 1 < SC_J)
            def _():
                pltpu.make_async_copy(rows[0], acc_sh.at[sidx_v.at[j0]],
                                      ssems[0]).wait()
                gather(j1 + 1, 0, tab_h)

            return carry

        lax.fori_loop(0, SC_J // 2, jbody, 0)
        pltpu.make_async_copy(rows[0], acc_sh.at[sidx_v.at[0]],
                              ssems[0]).wait()
        pltpu.make_async_copy(rows[1], acc_sh.at[sidx_v.at[0]],
                              ssems[1]).wait()
        plsc.subcore_barrier()
        pltpu.sync_copy(acc_sh.at[pl.ds(fl0, flr)],
                        out_h.at[cid].at[pl.ds(fl0, flr)])
        plsc.subcore_barrier()


def _call_sc(tab_in, tab_out, sg, ss, dg, ds_, zeros_nd):
    mesh = plsc.VectorSubcoreMesh(core_axis_name="c", subcore_axis_name="s")
    fn = pl.kernel(
        _sc_body,
        out_type=[jax.ShapeDtypeStruct((2, N, 128), jnp.float32)] * 2,
        mesh=mesh,
        scratch_types=[
            pltpu.VMEM((SC_J, SC_C), jnp.int32),
            pltpu.VMEM((SC_J, SC_C), jnp.int32),
            pltpu.VMEM((SC_C, 128), jnp.float32),
            pltpu.VMEM((SC_C, 128), jnp.float32),
            pltpu.VMEM_SHARED((N_ACC, 128), jnp.float32),
            pltpu.SemaphoreType.DMA,
            pltpu.SemaphoreType.DMA,
            pltpu.SemaphoreType.DMA,
            pltpu.SemaphoreType.DMA,
        ],
    )
    return fn(tab_in, tab_out, sg, ss, dg, ds_, zeros_nd)


def _k2a_body(proj_ref, inc_ref, outg_ref,
              g1_ref, be1_ref, w2_ref, b2_ref, g2_ref, be2_ref,
              wk_ref, bk_ref, wv_ref,
              xn_ref, ktv_ref):
    n = pl.program_id(0)
    inc = inc_ref[0] + inc_ref[1]
    outg = outg_ref[0] + outg_ref[1]
    for l in range(L):
        agg_in = inc[:, l * 32:(l + 1) * 32]
        agg_out = outg[:, l * 32:(l + 1) * 32]
        out = jnp.concatenate([proj_ref[l], agg_in, agg_out], axis=-1)
        h1 = _ln(_silu(out), g1_ref[...], be1_ref[...])
        h2 = h1 @ w2_ref[...] + b2_ref[...]
        xn = _ln(_silu(h2), g2_ref[...], be2_ref[...])
        xn_ref[l] = xn

        k = _elu1(xn @ wk_ref[...] + bk_ref[...])
        v = xn @ wv_ref[...]
        ktv = lax.dot_general(k, v, (((0,), (0,)), ((), ())),
                              preferred_element_type=jnp.float32)

        @pl.when(n == 0)
        def _():
            ktv_ref[l] = ktv

        @pl.when(n != 0)
        def _():
            ktv_ref[l] += ktv


def _call_k2a(proj, incP, outgP, lp):
    f, ln_ = lp['feat'], lp['lin']
    part = pl.BlockSpec((2, BN, 128), lambda n: (0, n, 0))
    return pl.pallas_call(
        _k2a_body,
        grid=(NB,),
        in_specs=[
            _xspec(128), part, part,
            _full_spec((1, 192)), _full_spec((1, 192)),
            _full_spec((192, 128)), _full_spec((1, 128)),
            _full_spec((1, 128)), _full_spec((1, 128)),
            _full_spec((128, 64)), _full_spec((1, 64)),
            _full_spec((128, 32)),
        ],
        out_specs=[
            _xspec(128),
            pl.BlockSpec((L, 64, 32), lambda n: (0, 0, 0)),
        ],
        out_shape=[
            jax.ShapeDtypeStruct((L, N, 128), jnp.float32),
            jax.ShapeDtypeStruct((L, 64, 32), jnp.float32),
        ],
    )(proj, incP, outgP,
      f['g1'][None], f['be1'][None], f['W2'], f['b2'][None],
      f['g2'][None], f['be2'][None],
      ln_['Wk'], ln_['bk'][None], ln_['Wv'])


def _k2b_body(peproj_ref, pei_ref, peo_ref, pw2_ref, pb2_ref, pen_ref):
    pei = pei_ref[0] + pei_ref[1]
    peo = peo_ref[0] + peo_ref[1]
    for l in range(L):
        pe_in = pei[:, l * 32:(l + 1) * 32]
        pe_out = peo[:, l * 32:(l + 1) * 32]
        pout = jnp.concatenate([peproj_ref[l], pe_in, pe_out], axis=-1)
        pen_ref[l] = jnp.tanh(jnp.tanh(pout) @ pw2_ref[...] + pb2_ref[...])


def _call_k2b(peproj, peincP, peoutgP, lp):
    pp = lp['pe']
    part = pl.BlockSpec((2, BN, 128), lambda n: (0, n, 0))
    return pl.pallas_call(
        _k2b_body,
        grid=(NB,),
        in_specs=[
            _xspec(32), part, part,
            _full_spec((96, 32)), _full_spec((1, 32)),
        ],
        out_specs=_xspec(32),
        out_shape=jax.ShapeDtypeStruct((L, N, 32), jnp.float32),
    )(peproj, peincP, peoutgP, pp['W2'], pp['b2'][None])


def _k3_body(xn_ref, ktv_ref, wq_ref, bq_ref, wc_x_ref, wc_a_ref, bc_ref,
             x_ref):
    for l in range(L):
        xn = xn_ref[l]
        q = _elu1(xn @ wq_ref[...] + bq_ref[...])
        att = jnp.dot(q, ktv_ref[l], preferred_element_type=jnp.float32)
        x_ref[l] = _silu(xn @ wc_x_ref[...] + att @ wc_a_ref[...]
                         + bc_ref[...])


def _call_k3(x_new, ktv, lp):
    ln_, cb = lp['lin'], lp['comb']
    wc = cb['Wc']
    return pl.pallas_call(
        _k3_body,
        grid=(NB,),
        in_specs=[
            _xspec(128),
            pl.BlockSpec((L, 64, 32), lambda n: (0, 0, 0)),
            _full_spec((128, 64)), _full_spec((1, 64)),
            _full_spec((128, 128)), _full_spec((32, 128)),
            _full_spec((1, 128)),
        ],
        out_specs=_xspec(128),
        out_shape=jax.ShapeDtypeStruct((L, N, 128), jnp.float32),
    )(x_new, ktv, ln_['Wq'], ln_['bq'][None], wc[:128], wc[128:],
      cb['bc'][None])


def kernel(op_code, features, edge_index, lengths, params):
    p = params
    normal = features[..., :127]
    pe_feat = features[..., 127:]
    opc = op_code.astype(jnp.float32)[..., None]

    src, dst = edge_index[0], edge_index[1]
    pad = E_PAD - E
    zpad = jnp.arange(pad, dtype=jnp.int32) % 128
    npad = N + (jnp.arange(pad, dtype=jnp.int32) % 128)
    sg = jnp.concatenate([src, zpad]).reshape(IDX_ROWS, SC_C)
    ss = jnp.concatenate([src, npad]).reshape(IDX_ROWS, SC_C)
    dg = jnp.concatenate([dst, zpad]).reshape(IDX_ROWS, SC_C)
    ds_ = jnp.concatenate([dst, npad]).reshape(IDX_ROWS, SC_C)
    zeros_nd = jnp.zeros((N, 128), jnp.float32)

    x, pe_emb = _call_k0(opc, normal, pe_feat, p)
    for lp in p['layers']:
        proj, peproj, pin_t, pout_t, pepin_t, pepout_t = _call_k1(pe_emb, x, lp)
        incP, outgP = _call_sc(pin_t, pout_t, sg, ss, dg, ds_, zeros_nd)
        peincP, peoutgP = _call_sc(pepin_t, pepout_t, sg, ss, dg, ds_,
                                   zeros_nd)
        x_new, ktv = _call_k2a(proj, incP, outgP, lp)
        x = _call_k3(x_new, ktv, lp)
        pe_emb = _call_k2b(peproj, peincP, peoutgP, lp)
    return x


# speedup vs baseline: 3.7320x; 1.1175x over previous
import functools

import jax
import jax.numpy as jnp
from jax import lax
from jax.experimental import pallas as pl
from jax.experimental.pallas import tpu as pltpu
from jax.experimental.pallas import tpu_sc as plsc

L = 4
N = 10000
E = 160000
BN = 1000
NB = N // BN

SC_C = 128
SC_J = 40
E_PAD = 32 * SC_J * SC_C
IDX_ROWS = E_PAD // SC_C
N_ACC = N + 128


def _silu(x):
    return x * jax.nn.sigmoid(x)


def _ln(h, g, b):
    m = jnp.mean(h, axis=-1, keepdims=True)
    d = h - m
    v = jnp.mean(d * d, axis=-1, keepdims=True)
    return d * lax.rsqrt(v + 1e-5) * g + b


def _elu1(z):
    return jnp.where(z > 0, z + 1.0, jnp.exp(jnp.minimum(z, 0.0)))


def _full_spec(shape):
    nd = len(shape)
    return pl.BlockSpec(shape, lambda n, _nd=nd: (0,) * _nd)


def _xspec(d):
    return pl.BlockSpec((L, BN, d), lambda n: (0, n, 0))


def _k0_body(opc_ref, normal_ref, pe_ref, emb_ref, win_e_ref, win_n_ref,
             bin_ref, gin_ref, bein_ref, wpe_ref, bpe_ref,
             x_ref, peemb_ref):
    iota = lax.broadcasted_iota(jnp.int32, (BN, 128), 1).astype(jnp.float32)
    for l in range(L):
        opc = opc_ref[l]
        oh = jnp.where(opc == iota, 1.0, 0.0)
        emb = jnp.dot(oh, emb_ref[...], preferred_element_type=jnp.float32)
        pre = (emb @ win_e_ref[...] + normal_ref[l] @ win_n_ref[...]
               + bin_ref[...])
        x_ref[l] = _ln(_silu(pre), gin_ref[...], bein_ref[...])
        peemb_ref[l] = jnp.tanh(pe_ref[l] @ wpe_ref[...] + bpe_ref[...])


def _call_k0(opc, normal, pe_feat, p):
    win = p['W_in']
    return pl.pallas_call(
        _k0_body,
        grid=(NB,),
        in_specs=[
            pl.BlockSpec((L, BN, 1), lambda n: (0, n, 0)),
            pl.BlockSpec((L, BN, 127), lambda n: (0, n, 0)),
            _xspec(16),
            _full_spec((128, 32)),
            _full_spec((32, 128)),
            _full_spec((127, 128)),
            _full_spec((1, 128)),
            _full_spec((1, 128)),
            _full_spec((1, 128)),
            _full_spec((16, 32)),
            _full_spec((1, 32)),
        ],
        out_specs=[_xspec(128), _xspec(32)],
        out_shape=[
            jax.ShapeDtypeStruct((L, N, 128), jnp.float32),
            jax.ShapeDtypeStruct((L, N, 32), jnp.float32),
        ],
    )(opc, normal, pe_feat, p['emb'], win[:32], win[32:],
      p['b_in'][None], p['g_in'][None], p['be_in'][None],
      p['W_pe'], p['b_pe'][None])


def _k1_body(pe_ref, x_ref, w_pe_ref, w_x_ref, b_ref, wi_pe_ref, wi_x_ref,
             bi_ref, wo_pe_ref, wo_x_ref, bo_ref,
             pw_ref, pb_ref, pwi_ref, pbi_ref, pwo_ref, pbo_ref,
             proj_ref, peproj_ref, pin_ref, pout_ref, pepin_ref, pepout_ref):
    pin, pout, pepin, pepout = [], [], [], []
    for l in range(L):
        pe = pe_ref[l]
        x = x_ref[l]
        proj_ref[l] = pe @ w_pe_ref[...] + x @ w_x_ref[...] + b_ref[...]
        pin.append(pe @ wi_pe_ref[...] + x @ wi_x_ref[...] + bi_ref[...])
        pout.append(pe @ wo_pe_ref[...] + x @ wo_x_ref[...] + bo_ref[...])
        peproj_ref[l] = pe @ pw_ref[...] + pb_ref[...]
        pepin.append(pe @ pwi_ref[...] + pbi_ref[...])
        pepout.append(pe @ pwo_ref[...] + pbo_ref[...])
    pin_ref[...] = jnp.concatenate(pin, axis=-1)
    pout_ref[...] = jnp.concatenate(pout, axis=-1)
    pepin_ref[...] = jnp.concatenate(pepin, axis=-1)
    pepout_ref[...] = jnp.concatenate(pepout, axis=-1)


def _call_k1(pe_emb, x, lp):
    f, pp = lp['feat'], lp['pe']
    tab_spec = pl.BlockSpec((BN, 128), lambda n: (n, 0))
    tab_shape = jax.ShapeDtypeStruct((N, 128), jnp.float32)
    return pl.pallas_call(
        _k1_body,
        grid=(NB,),
        in_specs=[
            _xspec(32),
            _xspec(128),
            _full_spec((32, 128)), _full_spec((128, 128)), _full_spec((1, 128)),
            _full_spec((32, 32)), _full_spec((128, 32)), _full_spec((1, 32)),
            _full_spec((32, 32)), _full_spec((128, 32)), _full_spec((1, 32)),
            _full_spec((32, 32)), _full_spec((1, 32)),
            _full_spec((32, 32)), _full_spec((1, 32)),
            _full_spec((32, 32)), _full_spec((1, 32)),
        ],
        out_specs=[
            _xspec(128), _xspec(32),
            tab_spec, tab_spec, tab_spec, tab_spec,
        ],
        out_shape=[
            jax.ShapeDtypeStruct((L, N, 128), jnp.float32),
            jax.ShapeDtypeStruct((L, N, 32), jnp.float32),
            tab_shape, tab_shape, tab_shape, tab_shape,
        ],
    )(pe_emb, x,
      f['W'][:32], f['W'][32:], f['b'][None],
      f['Wi'][:32], f['Wi'][32:], f['bi'][None],
      f['Wo'][:32], f['Wo'][32:], f['bo'][None],
      pp['W'], pp['b'][None], pp['Wi'], pp['bi'][None],
      pp['Wo'], pp['bo'][None])


def _sc_body(tin_h, tout_h, sg_h, ss_h, dg_h, ds_h,
             zeros_h, dep_h, inc_h, outg_h,
             gidx_v, sidx_v, rows0_v, rows1_v, acc_sh,
             gsem0, gsem1, ssem0, ssem1):
    cid = lax.axis_index("c")
    sid = lax.axis_index("s")
    idx_row0 = cid * (IDX_ROWS // 2) + sid * SC_J
    fl0 = sid * 624
    flr = 640
    rows = (rows0_v, rows1_v)
    gsems = (gsem0, gsem1)
    ssems = (ssem0, ssem1)

    def gather(j, b, tab_h):
        pltpu.async_copy(tab_h.at[gidx_v.at[j]], rows[b], gsems[b])

    def scatter(j, b):
        pltpu.async_copy(rows[b], acc_sh.at[sidx_v.at[j]], ssems[b],
                         add=True)

    for tab_h, gh, sh, out_h in (
            (tin_h, sg_h, ds_h, inc_h),
            (tout_h, dg_h, ss_h, outg_h)):
        pltpu.sync_copy(zeros_h.at[pl.ds(fl0, flr)], acc_sh.at[pl.ds(fl0, flr)])
        pltpu.sync_copy(gh.at[pl.ds(idx_row0, SC_J)], gidx_v)
        pltpu.sync_copy(sh.at[pl.ds(idx_row0, SC_J)], sidx_v)
        plsc.subcore_barrier()

        gather(0, 0, tab_h)

        def jbody(i, carry):
            j0 = 2 * i
            j1 = 2 * i + 1
            pltpu.make_async_copy(tab_h.at[gidx_v.at[j0]], rows[0],
                                  gsems[0]).wait()
            scatter(j0, 0)

            @pl.when(i > 0)
            def _():
                pltpu.make_async_copy(rows[1], acc_sh.at[sidx_v.at[j1]],
                                      ssems[1]).wait()

            gather(j1, 1, tab_h)
            pltpu.make_async_copy(tab_h.at[gidx_v.at[j1]], rows[1],
                                  gsems[1]).wait()
            scatter(j1, 1)

            @pl.when(j1 + 1 < SC_J)
            def _():
                pltpu.make_async_copy(rows[0], acc_sh.at[sidx_v.at[j0]],
                                      ssems[0]).wait()
                gather(j1 + 1, 0, tab_h)

            return carry

        lax.fori_loop(0, SC_J // 2, jbody, 0)
        pltpu.make_async_copy(rows[0], acc_sh.at[sidx_v.at[0]],
                              ssems[0]).wait()
        pltpu.make_async_copy(rows[1], acc_sh.at[sidx_v.at[0]],
                              ssems[1]).wait()
        plsc.subcore_barrier()
        pltpu.sync_copy(acc_sh.at[pl.ds(fl0, flr)],
                        out_h.at[cid].at[pl.ds(fl0, flr)])
        plsc.subcore_barrier()


def _call_sc(tab_in, tab_out, sg, ss, dg, ds_, zeros_nd, dep):
    mesh = plsc.VectorSubcoreMesh(core_axis_name="c", subcore_axis_name="s")
    fn = pl.kernel(
        _sc_body,
        out_type=[jax.ShapeDtypeStruct((2, N, 128), jnp.float32)] * 2,
        mesh=mesh,
        scratch_types=[
            pltpu.VMEM((SC_J, SC_C), jnp.int32),
            pltpu.VMEM((SC_J, SC_C), jnp.int32),
            pltpu.VMEM((SC_C, 128), jnp.float32),
            pltpu.VMEM((SC_C, 128), jnp.float32),
            pltpu.VMEM_SHARED((N_ACC, 128), jnp.float32),
            pltpu.SemaphoreType.DMA,
            pltpu.SemaphoreType.DMA,
            pltpu.SemaphoreType.DMA,
            pltpu.SemaphoreType.DMA,
        ],
    )
    return fn(tab_in, tab_out, sg, ss, dg, ds_, zeros_nd, dep)


def _k2a_body(proj_ref, inc_ref, outg_ref,
              g1_ref, be1_ref, w2_ref, b2_ref, g2_ref, be2_ref,
              wk_ref, bk_ref, wv_ref,
              xn_ref, ktv_ref):
    n = pl.program_id(0)
    inc = inc_ref[0] + inc_ref[1]
    outg = outg_ref[0] + outg_ref[1]
    for l in range(L):
        agg_in = inc[:, l * 32:(l + 1) * 32]
        agg_out = outg[:, l * 32:(l + 1) * 32]
        out = jnp.concatenate([proj_ref[l], agg_in, agg_out], axis=-1)
        h1 = _ln(_silu(out), g1_ref[...], be1_ref[...])
        h2 = h1 @ w2_ref[...] + b2_ref[...]
        xn = _ln(_silu(h2), g2_ref[...], be2_ref[...])
        xn_ref[l] = xn

        k = _elu1(xn @ wk_ref[...] + bk_ref[...])
        v = xn @ wv_ref[...]
        ktv = lax.dot_general(k, v, (((0,), (0,)), ((), ())),
                              preferred_element_type=jnp.float32)

        @pl.when(n == 0)
        def _():
            ktv_ref[l] = ktv

        @pl.when(n != 0)
        def _():
            ktv_ref[l] += ktv


def _call_k2a(proj, incP, outgP, lp):
    f, ln_ = lp['feat'], lp['lin']
    part = pl.BlockSpec((2, BN, 128), lambda n: (0, n, 0))
    return pl.pallas_call(
        _k2a_body,
        grid=(NB,),
        in_specs=[
            _xspec(128), part, part,
            _full_spec((1, 192)), _full_spec((1, 192)),
            _full_spec((192, 128)), _full_spec((1, 128)),
            _full_spec((1, 128)), _full_spec((1, 128)),
            _full_spec((128, 64)), _full_spec((1, 64)),
            _full_spec((128, 32)),
        ],
        out_specs=[
            _xspec(128),
            pl.BlockSpec((L, 64, 32), lambda n: (0, 0, 0)),
        ],
        out_shape=[
            jax.ShapeDtypeStruct((L, N, 128), jnp.float32),
            jax.ShapeDtypeStruct((L, 64, 32), jnp.float32),
        ],
    )(proj, incP, outgP,
      f['g1'][None], f['be1'][None], f['W2'], f['b2'][None],
      f['g2'][None], f['be2'][None],
      ln_['Wk'], ln_['bk'][None], ln_['Wv'])


def _k2b_body(peproj_ref, pei_ref, peo_ref, pw2_ref, pb2_ref, pen_ref):
    pei = pei_ref[0] + pei_ref[1]
    peo = peo_ref[0] + peo_ref[1]
    for l in range(L):
        pe_in = pei[:, l * 32:(l + 1) * 32]
        pe_out = peo[:, l * 32:(l + 1) * 32]
        pout = jnp.concatenate([peproj_ref[l], pe_in, pe_out], axis=-1)
        pen_ref[l] = jnp.tanh(jnp.tanh(pout) @ pw2_ref[...] + pb2_ref[...])


def _call_k2b(peproj, peincP, peoutgP, lp):
    pp = lp['pe']
    part = pl.BlockSpec((2, BN, 128), lambda n: (0, n, 0))
    return pl.pallas_call(
        _k2b_body,
        grid=(NB,),
        in_specs=[
            _xspec(32), part, part,
            _full_spec((96, 32)), _full_spec((1, 32)),
        ],
        out_specs=_xspec(32),
        out_shape=jax.ShapeDtypeStruct((L, N, 32), jnp.float32),
    )(peproj, peincP, peoutgP, pp['W2'], pp['b2'][None])


def _k3_body(xn_ref, ktv_ref, wq_ref, bq_ref, wc_x_ref, wc_a_ref, bc_ref,
             x_ref):
    for l in range(L):
        xn = xn_ref[l]
        q = _elu1(xn @ wq_ref[...] + bq_ref[...])
        att = jnp.dot(q, ktv_ref[l], preferred_element_type=jnp.float32)
        x_ref[l] = _silu(xn @ wc_x_ref[...] + att @ wc_a_ref[...]
                         + bc_ref[...])


def _call_k3(x_new, ktv, lp):
    ln_, cb = lp['lin'], lp['comb']
    wc = cb['Wc']
    return pl.pallas_call(
        _k3_body,
        grid=(NB,),
        in_specs=[
            _xspec(128),
            pl.BlockSpec((L, 64, 32), lambda n: (0, 0, 0)),
            _full_spec((128, 64)), _full_spec((1, 64)),
            _full_spec((128, 128)), _full_spec((32, 128)),
            _full_spec((1, 128)),
        ],
        out_specs=_xspec(128),
        out_shape=jax.ShapeDtypeStruct((L, N, 128), jnp.float32),
    )(x_new, ktv, ln_['Wq'], ln_['bq'][None], wc[:128], wc[128:],
      cb['bc'][None])


def kernel(op_code, features, edge_index, lengths, params):
    p = params
    normal = features[..., :127]
    pe_feat = features[..., 127:]
    opc = op_code.astype(jnp.float32)[..., None]

    src, dst = edge_index[0], edge_index[1]
    pad = E_PAD - E
    zpad = jnp.arange(pad, dtype=jnp.int32) % 128
    npad = N + (jnp.arange(pad, dtype=jnp.int32) % 128)
    sg = jnp.concatenate([src, zpad]).reshape(IDX_ROWS, SC_C)
    ss = jnp.concatenate([src, npad]).reshape(IDX_ROWS, SC_C)
    dg = jnp.concatenate([dst, zpad]).reshape(IDX_ROWS, SC_C)
    ds_ = jnp.concatenate([dst, npad]).reshape(IDX_ROWS, SC_C)
    zeros_nd = jnp.zeros((N, 128), jnp.float32)

    x, pe_emb = _call_k0(opc, normal, pe_feat, p)
    for lp in p['layers']:
        proj, peproj, pin_t, pout_t, pepin_t, pepout_t = _call_k1(pe_emb, x, lp)
        incP, outgP = _call_sc(pin_t, pout_t, sg, ss, dg, ds_, zeros_nd,
                               zeros_nd)
        peincP, peoutgP = _call_sc(pepin_t, pepout_t, sg, ss, dg, ds_,
                                   zeros_nd, incP)
        x_new, ktv = _call_k2a(proj, incP, outgP, lp)
        x = _call_k3(x_new, ktv, lp)
        pe_emb = _call_k2b(peproj, peincP, peoutgP, lp)
    return x


# speedup vs baseline: 3.8543x; 1.0327x over previous
import functools

import jax
import jax.numpy as jnp
from jax import lax
from jax.experimental import pallas as pl
from jax.experimental.pallas import tpu as pltpu
from jax.experimental.pallas import tpu_sc as plsc

L = 4
N = 10000
E = 160000
BN = 1000
NB = N // BN

SC_C = 128
SC_J = 40
E_PAD = 32 * SC_J * SC_C
IDX_ROWS = E_PAD // SC_C
N_ACC = N + 128


def _silu(x):
    return x * jax.nn.sigmoid(x)


def _ln(h, g, b):
    m = jnp.mean(h, axis=-1, keepdims=True)
    d = h - m
    v = jnp.mean(d * d, axis=-1, keepdims=True)
    return d * lax.rsqrt(v + 1e-5) * g + b


def _elu1(z):
    return jnp.where(z > 0, z + 1.0, jnp.exp(jnp.minimum(z, 0.0)))


def _full_spec(shape):
    nd = len(shape)
    return pl.BlockSpec(shape, lambda n, _nd=nd: (0,) * _nd)


def _xspec(d):
    return pl.BlockSpec((L, BN, d), lambda n: (0, n, 0))


def _k0_body(opc_ref, normal_ref, pe_ref, emb_ref, win_e_ref, win_n_ref,
             bin_ref, gin_ref, bein_ref, wpe_ref, bpe_ref,
             x_ref, peemb_ref):
    iota = lax.broadcasted_iota(jnp.int32, (BN, 128), 1).astype(jnp.float32)
    for l in range(L):
        opc = opc_ref[l]
        oh = jnp.where(opc == iota, 1.0, 0.0)
        emb = jnp.dot(oh, emb_ref[...], preferred_element_type=jnp.float32)
        pre = (emb @ win_e_ref[...] + normal_ref[l] @ win_n_ref[...]
               + bin_ref[...])
        x_ref[l] = _ln(_silu(pre), gin_ref[...], bein_ref[...])
        peemb_ref[l] = jnp.tanh(pe_ref[l] @ wpe_ref[...] + bpe_ref[...])


def _call_k0(opc, normal, pe_feat, p):
    win = p['W_in']
    return pl.pallas_call(
        _k0_body,
        grid=(NB,),
        in_specs=[
            pl.BlockSpec((L, BN, 1), lambda n: (0, n, 0)),
            pl.BlockSpec((L, BN, 127), lambda n: (0, n, 0)),
            _xspec(16),
            _full_spec((128, 32)),
            _full_spec((32, 128)),
            _full_spec((127, 128)),
            _full_spec((1, 128)),
            _full_spec((1, 128)),
            _full_spec((1, 128)),
            _full_spec((16, 32)),
            _full_spec((1, 32)),
        ],
        out_specs=[_xspec(128), _xspec(32)],
        out_shape=[
            jax.ShapeDtypeStruct((L, N, 128), jnp.float32),
            jax.ShapeDtypeStruct((L, N, 32), jnp.float32),
        ],
    )(opc, normal, pe_feat, p['emb'], win[:32], win[32:],
      p['b_in'][None], p['g_in'][None], p['be_in'][None],
      p['W_pe'], p['b_pe'][None])


def _k1a_body(pe_ref, x_ref, wi_pe_ref, wi_x_ref, bi_ref,
              wo_pe_ref, wo_x_ref, bo_ref, pin_ref, pout_ref):
    pin, pout = [], []
    for l in range(L):
        pe = pe_ref[l]
        x = x_ref[l]
        pin.append(pe @ wi_pe_ref[...] + x @ wi_x_ref[...] + bi_ref[...])
        pout.append(pe @ wo_pe_ref[...] + x @ wo_x_ref[...] + bo_ref[...])
    pin_ref[...] = jnp.concatenate(pin, axis=-1)
    pout_ref[...] = jnp.concatenate(pout, axis=-1)


def _call_k1a(pe_emb, x, lp):
    f = lp['feat']
    tab_spec = pl.BlockSpec((BN, 128), lambda n: (n, 0))
    tab_shape = jax.ShapeDtypeStruct((N, 128), jnp.float32)
    return pl.pallas_call(
        _k1a_body,
        grid=(NB,),
        in_specs=[
            _xspec(32),
            _xspec(128),
            _full_spec((32, 32)), _full_spec((128, 32)), _full_spec((1, 32)),
            _full_spec((32, 32)), _full_spec((128, 32)), _full_spec((1, 32)),
        ],
        out_specs=[tab_spec, tab_spec],
        out_shape=[tab_shape, tab_shape],
    )(pe_emb, x,
      f['Wi'][:32], f['Wi'][32:], f['bi'][None],
      f['Wo'][:32], f['Wo'][32:], f['bo'][None])


def _k1b_body(pe_ref, x_ref, w_pe_ref, w_x_ref, b_ref,
              pw_ref, pb_ref, pwi_ref, pbi_ref, pwo_ref, pbo_ref,
              proj_ref, peproj_ref, pepin_ref, pepout_ref):
    pepin, pepout = [], []
    for l in range(L):
        pe = pe_ref[l]
        x = x_ref[l]
        proj_ref[l] = pe @ w_pe_ref[...] + x @ w_x_ref[...] + b_ref[...]
        peproj_ref[l] = pe @ pw_ref[...] + pb_ref[...]
        pepin.append(pe @ pwi_ref[...] + pbi_ref[...])
        pepout.append(pe @ pwo_ref[...] + pbo_ref[...])
    pepin_ref[...] = jnp.concatenate(pepin, axis=-1)
    pepout_ref[...] = jnp.concatenate(pepout, axis=-1)


def _call_k1b(pe_emb, x, lp):
    f, pp = lp['feat'], lp['pe']
    tab_spec = pl.BlockSpec((BN, 128), lambda n: (n, 0))
    tab_shape = jax.ShapeDtypeStruct((N, 128), jnp.float32)
    return pl.pallas_call(
        _k1b_body,
        grid=(NB,),
        in_specs=[
            _xspec(32),
            _xspec(128),
            _full_spec((32, 128)), _full_spec((128, 128)), _full_spec((1, 128)),
            _full_spec((32, 32)), _full_spec((1, 32)),
            _full_spec((32, 32)), _full_spec((1, 32)),
            _full_spec((32, 32)), _full_spec((1, 32)),
        ],
        out_specs=[
            _xspec(128), _xspec(32), tab_spec, tab_spec,
        ],
        out_shape=[
            jax.ShapeDtypeStruct((L, N, 128), jnp.float32),
            jax.ShapeDtypeStruct((L, N, 32), jnp.float32),
            tab_shape, tab_shape,
        ],
    )(pe_emb, x,
      f['W'][:32], f['W'][32:], f['b'][None],
      pp['W'], pp['b'][None], pp['Wi'], pp['bi'][None],
      pp['Wo'], pp['bo'][None])


def _sc_body(tin_h, tout_h, sg_h, ss_h, dg_h, ds_h,
             zeros_h, dep_h, inc_h, outg_h,
             gidx_v, sidx_v, rows0_v, rows1_v, acc_sh,
             gsem0, gsem1, ssem0, ssem1):
    cid = lax.axis_index("c")
    sid = lax.axis_index("s")
    idx_row0 = cid * (IDX_ROWS // 2) + sid * SC_J
    fl0 = sid * 624
    flr = 640
    rows = (rows0_v, rows1_v)
    gsems = (gsem0, gsem1)
    ssems = (ssem0, ssem1)

    def gather(j, b, tab_h):
        pltpu.async_copy(tab_h.at[gidx_v.at[j]], rows[b], gsems[b])

    def scatter(j, b):
        pltpu.async_copy(rows[b], acc_sh.at[sidx_v.at[j]], ssems[b],
                         add=True)

    for tab_h, gh, sh, out_h in (
            (tin_h, sg_h, ds_h, inc_h),
            (tout_h, dg_h, ss_h, outg_h)):
        pltpu.sync_copy(zeros_h.at[pl.ds(fl0, flr)], acc_sh.at[pl.ds(fl0, flr)])
        pltpu.sync_copy(gh.at[pl.ds(idx_row0, SC_J)], gidx_v)
        pltpu.sync_copy(sh.at[pl.ds(idx_row0, SC_J)], sidx_v)
        plsc.subcore_barrier()

        gather(0, 0, tab_h)

        def jbody(i, carry):
            j0 = 2 * i
            j1 = 2 * i + 1
            pltpu.make_async_copy(tab_h.at[gidx_v.at[j0]], rows[0],
                                  gsems[0]).wait()
            scatter(j0, 0)

            @pl.when(i > 0)
            def _():
                pltpu.make_async_copy(rows[1], acc_sh.at[sidx_v.at[j1]],
                                      ssems[1]).wait()

            gather(j1, 1, tab_h)
            pltpu.make_async_copy(tab_h.at[gidx_v.at[j1]], rows[1],
                                  gsems[1]).wait()
            scatter(j1, 1)

            @pl.when(j1 + 1 < SC_J)
            def _():
                pltpu.make_async_copy(rows[0], acc_sh.at[sidx_v.at[j0]],
                                      ssems[0]).wait()
                gather(j1 + 1, 0, tab_h)

            return carry

        lax.fori_loop(0, SC_J // 2, jbody, 0)
        pltpu.make_async_copy(rows[0], acc_sh.at[sidx_v.at[0]],
                              ssems[0]).wait()
        pltpu.make_async_copy(rows[1], acc_sh.at[sidx_v.at[0]],
                              ssems[1]).wait()
        plsc.subcore_barrier()
        pltpu.sync_copy(acc_sh.at[pl.ds(fl0, flr)],
                        out_h.at[cid].at[pl.ds(fl0, flr)])
        plsc.subcore_barrier()


def _call_sc(tab_in, tab_out, sg, ss, dg, ds_, zeros_nd, dep):
    mesh = plsc.VectorSubcoreMesh(core_axis_name="c", subcore_axis_name="s")
    fn = pl.kernel(
        _sc_body,
        out_type=[jax.ShapeDtypeStruct((2, N, 128), jnp.float32)] * 2,
        mesh=mesh,
        scratch_types=[
            pltpu.VMEM((SC_J, SC_C), jnp.int32),
            pltpu.VMEM((SC_J, SC_C), jnp.int32),
            pltpu.VMEM((SC_C, 128), jnp.float32),
            pltpu.VMEM((SC_C, 128), jnp.float32),
            pltpu.VMEM_SHARED((N_ACC, 128), jnp.float32),
            pltpu.SemaphoreType.DMA,
            pltpu.SemaphoreType.DMA,
            pltpu.SemaphoreType.DMA,
            pltpu.SemaphoreType.DMA,
        ],
    )
    return fn(tab_in, tab_out, sg, ss, dg, ds_, zeros_nd, dep)


def _k2a_body(proj_ref, inc_ref, outg_ref,
              g1_ref, be1_ref, w2_ref, b2_ref, g2_ref, be2_ref,
              wk_ref, bk_ref, wv_ref,
              xn_ref, ktv_ref):
    n = pl.program_id(0)
    inc = inc_ref[0] + inc_ref[1]
    outg = outg_ref[0] + outg_ref[1]
    for l in range(L):
        agg_in = inc[:, l * 32:(l + 1) * 32]
        agg_out = outg[:, l * 32:(l + 1) * 32]
        out = jnp.concatenate([proj_ref[l], agg_in, agg_out], axis=-1)
        h1 = _ln(_silu(out), g1_ref[...], be1_ref[...])
        h2 = h1 @ w2_ref[...] + b2_ref[...]
        xn = _ln(_silu(h2), g2_ref[...], be2_ref[...])
        xn_ref[l] = xn

        k = _elu1(xn @ wk_ref[...] + bk_ref[...])
        v = xn @ wv_ref[...]
        ktv = lax.dot_general(k, v, (((0,), (0,)), ((), ())),
                              preferred_element_type=jnp.float32)

        @pl.when(n == 0)
        def _():
            ktv_ref[l] = ktv

        @pl.when(n != 0)
        def _():
            ktv_ref[l] += ktv


def _call_k2a(proj, incP, outgP, lp):
    f, ln_ = lp['feat'], lp['lin']
    part = pl.BlockSpec((2, BN, 128), lambda n: (0, n, 0))
    return pl.pallas_call(
        _k2a_body,
        grid=(NB,),
        in_specs=[
            _xspec(128), part, part,
            _full_spec((1, 192)), _full_spec((1, 192)),
            _full_spec((192, 128)), _full_spec((1, 128)),
            _full_spec((1, 128)), _full_spec((1, 128)),
            _full_spec((128, 64)), _full_spec((1, 64)),
            _full_spec((128, 32)),
        ],
        out_specs=[
            _xspec(128),
            pl.BlockSpec((L, 64, 32), lambda n: (0, 0, 0)),
        ],
        out_shape=[
            jax.ShapeDtypeStruct((L, N, 128), jnp.float32),
            jax.ShapeDtypeStruct((L, 64, 32), jnp.float32),
        ],
    )(proj, incP, outgP,
      f['g1'][None], f['be1'][None], f['W2'], f['b2'][None],
      f['g2'][None], f['be2'][None],
      ln_['Wk'], ln_['bk'][None], ln_['Wv'])


def _k2b_body(peproj_ref, pei_ref, peo_ref, pw2_ref, pb2_ref, pen_ref):
    pei = pei_ref[0] + pei_ref[1]
    peo = peo_ref[0] + peo_ref[1]
    for l in range(L):
        pe_in = pei[:, l * 32:(l + 1) * 32]
        pe_out = peo[:, l * 32:(l + 1) * 32]
        pout = jnp.concatenate([peproj_ref[l], pe_in, pe_out], axis=-1)
        pen_ref[l] = jnp.tanh(jnp.tanh(pout) @ pw2_ref[...] + pb2_ref[...])


def _call_k2b(peproj, peincP, peoutgP, lp):
    pp = lp['pe']
    part = pl.BlockSpec((2, BN, 128), lambda n: (0, n, 0))
    return pl.pallas_call(
        _k2b_body,
        grid=(NB,),
        in_specs=[
            _xspec(32), part, part,
            _full_spec((96, 32)), _full_spec((1, 32)),
        ],
        out_specs=_xspec(32),
        out_shape=jax.ShapeDtypeStruct((L, N, 32), jnp.float32),
    )(peproj, peincP, peoutgP, pp['W2'], pp['b2'][None])


def _k3_body(xn_ref, ktv_ref, wq_ref, bq_ref, wc_x_ref, wc_a_ref, bc_ref,
             x_ref):
    for l in range(L):
        xn = xn_ref[l]
        q = _elu1(xn @ wq_ref[...] + bq_ref[...])
        att = jnp.dot(q, ktv_ref[l], preferred_element_type=jnp.float32)
        x_ref[l] = _silu(xn @ wc_x_ref[...] + att @ wc_a_ref[...]
                         + bc_ref[...])


def _call_k3(x_new, ktv, lp):
    ln_, cb = lp['lin'], lp['comb']
    wc = cb['Wc']
    return pl.pallas_call(
        _k3_body,
        grid=(NB,),
        in_specs=[
            _xspec(128),
            pl.BlockSpec((L, 64, 32), lambda n: (0, 0, 0)),
            _full_spec((128, 64)), _full_spec((1, 64)),
            _full_spec((128, 128)), _full_spec((32, 128)),
            _full_spec((1, 128)),
        ],
        out_specs=_xspec(128),
        out_shape=jax.ShapeDtypeStruct((L, N, 128), jnp.float32),
    )(x_new, ktv, ln_['Wq'], ln_['bq'][None], wc[:128], wc[128:],
      cb['bc'][None])


def kernel(op_code, features, edge_index, lengths, params):
    p = params
    normal = features[..., :127]
    pe_feat = features[..., 127:]
    opc = op_code.astype(jnp.float32)[..., None]

    src, dst = edge_index[0], edge_index[1]
    pad = E_PAD - E
    zpad = jnp.arange(pad, dtype=jnp.int32) % 128
    npad = N + (jnp.arange(pad, dtype=jnp.int32) % 128)
    sg = jnp.concatenate([src, zpad]).reshape(IDX_ROWS, SC_C)
    ss = jnp.concatenate([src, npad]).reshape(IDX_ROWS, SC_C)
    dg = jnp.concatenate([dst, zpad]).reshape(IDX_ROWS, SC_C)
    ds_ = jnp.concatenate([dst, npad]).reshape(IDX_ROWS, SC_C)
    zeros_nd = jnp.zeros((N, 128), jnp.float32)

    x, pe_emb = _call_k0(opc, normal, pe_feat, p)
    for lp in p['layers']:
        pin_t, pout_t = _call_k1a(pe_emb, x, lp)
        incP, outgP = _call_sc(pin_t, pout_t, sg, ss, dg, ds_, zeros_nd,
                               zeros_nd)
        proj, peproj, pepin_t, pepout_t = _call_k1b(pe_emb, x, lp)
        peincP, peoutgP = _call_sc(pepin_t, pepout_t, sg, ss, dg, ds_,
                                   zeros_nd, incP)
        x_new, ktv = _call_k2a(proj, incP, outgP, lp)
        x = _call_k3(x_new, ktv, lp)
        pe_emb = _call_k2b(peproj, peincP, peoutgP, lp)
    return x


# speedup vs baseline: 3.8957x; 1.0108x over previous
import functools

import jax
import jax.numpy as jnp
from jax import lax
from jax.experimental import pallas as pl
from jax.experimental.pallas import tpu as pltpu
from jax.experimental.pallas import tpu_sc as plsc

L = 4
N = 10000
E = 160000
BN = 1000
NB = N // BN

SC_C = 128
SC_J = 40
E_PAD = 32 * SC_J * SC_C
IDX_ROWS = E_PAD // SC_C
N_ACC = N + 128


def _silu(x):
    return x * jax.nn.sigmoid(x)


def _ln(h, g, b):
    m = jnp.mean(h, axis=-1, keepdims=True)
    d = h - m
    v = jnp.mean(d * d, axis=-1, keepdims=True)
    return d * lax.rsqrt(v + 1e-5) * g + b


def _elu1(z):
    return jnp.where(z > 0, z + 1.0, jnp.exp(jnp.minimum(z, 0.0)))


def _full_spec(shape):
    nd = len(shape)
    return pl.BlockSpec(shape, lambda n, _nd=nd: (0,) * _nd)


def _xspec(d):
    return pl.BlockSpec((L, BN, d), lambda n: (0, n, 0))


def _k0_body(opc_ref, feat_ref, emb_ref, win_e_ref, win_n_ref,
             bin_ref, gin_ref, bein_ref, wpe_ref, bpe_ref,
             x_ref, peemb_ref):
    iota = lax.broadcasted_iota(jnp.int32, (BN, 128), 1).astype(jnp.float32)
    for l in range(L):
        opc = opc_ref[l]
        oh = jnp.where(opc == iota, 1.0, 0.0)
        emb = jnp.dot(oh, emb_ref[...], preferred_element_type=jnp.float32)
        f = feat_ref[l]
        pre = emb @ win_e_ref[...] + f @ win_n_ref[...] + bin_ref[...]
        x_ref[l] = _ln(_silu(pre), gin_ref[...], bein_ref[...])
        peemb_ref[l] = jnp.tanh(f @ wpe_ref[...] + bpe_ref[...])


def _call_k0(opc, features, p):
    win = p['W_in']
    nf = features.shape[-1]
    wn_pad = jnp.concatenate(
        [win[32:], jnp.zeros((nf - 127, 128), jnp.float32)], axis=0)
    wpe_pad = jnp.concatenate(
        [jnp.zeros((127, 32), jnp.float32), p['W_pe']], axis=0)
    return pl.pallas_call(
        _k0_body,
        grid=(NB,),
        in_specs=[
            pl.BlockSpec((L, BN, 1), lambda n: (0, n, 0)),
            pl.BlockSpec((L, BN, nf), lambda n: (0, n, 0)),
            _full_spec((128, 32)),
            _full_spec((32, 128)),
            _full_spec((nf, 128)),
            _full_spec((1, 128)),
            _full_spec((1, 128)),
            _full_spec((1, 128)),
            _full_spec((nf, 32)),
            _full_spec((1, 32)),
        ],
        out_specs=[_xspec(128), _xspec(32)],
        out_shape=[
            jax.ShapeDtypeStruct((L, N, 128), jnp.float32),
            jax.ShapeDtypeStruct((L, N, 32), jnp.float32),
        ],
    )(opc, features, p['emb'], win[:32], wn_pad,
      p['b_in'][None], p['g_in'][None], p['be_in'][None],
      wpe_pad, p['b_pe'][None])


def _k1a_body(pe_ref, x_ref, wi_pe_ref, wi_x_ref, bi_ref,
              wo_pe_ref, wo_x_ref, bo_ref, pin_ref, pout_ref):
    pin, pout = [], []
    for l in range(L):
        pe = pe_ref[l]
        x = x_ref[l]
        pin.append(pe @ wi_pe_ref[...] + x @ wi_x_ref[...] + bi_ref[...])
        pout.append(pe @ wo_pe_ref[...] + x @ wo_x_ref[...] + bo_ref[...])
    pin_ref[...] = jnp.concatenate(pin, axis=-1)
    pout_ref[...] = jnp.concatenate(pout, axis=-1)


def _call_k1a(pe_emb, x, lp):
    f = lp['feat']
    tab_spec = pl.BlockSpec((BN, 128), lambda n: (n, 0))
    tab_shape = jax.ShapeDtypeStruct((N, 128), jnp.float32)
    return pl.pallas_call(
        _k1a_body,
        grid=(NB,),
        in_specs=[
            _xspec(32),
            _xspec(128),
            _full_spec((32, 32)), _full_spec((128, 32)), _full_spec((1, 32)),
            _full_spec((32, 32)), _full_spec((128, 32)), _full_spec((1, 32)),
        ],
        out_specs=[tab_spec, tab_spec],
        out_shape=[tab_shape, tab_shape],
    )(pe_emb, x,
      f['Wi'][:32], f['Wi'][32:], f['bi'][None],
      f['Wo'][:32], f['Wo'][32:], f['bo'][None])


def _k1b_body(pe_ref, x_ref, w_pe_ref, w_x_ref, b_ref,
              pw_ref, pb_ref, pwi_ref, pbi_ref, pwo_ref, pbo_ref,
              proj_ref, peproj_ref, pepin_ref, pepout_ref):
    pepin, pepout = [], []
    for l in range(L):
        pe = pe_ref[l]
        x = x_ref[l]
        proj_ref[l] = pe @ w_pe_ref[...] + x @ w_x_ref[...] + b_ref[...]
        peproj_ref[l] = pe @ pw_ref[...] + pb_ref[...]
        pepin.append(pe @ pwi_ref[...] + pbi_ref[...])
        pepout.append(pe @ pwo_ref[...] + pbo_ref[...])
    pepin_ref[...] = jnp.concatenate(pepin, axis=-1)
    pepout_ref[...] = jnp.concatenate(pepout, axis=-1)


def _call_k1b(pe_emb, x, lp):
    f, pp = lp['feat'], lp['pe']
    tab_spec = pl.BlockSpec((BN, 128), lambda n: (n, 0))
    tab_shape = jax.ShapeDtypeStruct((N, 128), jnp.float32)
    return pl.pallas_call(
        _k1b_body,
        grid=(NB,),
        in_specs=[
            _xspec(32),
            _xspec(128),
            _full_spec((32, 128)), _full_spec((128, 128)), _full_spec((1, 128)),
            _full_spec((32, 32)), _full_spec((1, 32)),
            _full_spec((32, 32)), _full_spec((1, 32)),
            _full_spec((32, 32)), _full_spec((1, 32)),
        ],
        out_specs=[
            _xspec(128), _xspec(32), tab_spec, tab_spec,
        ],
        out_shape=[
            jax.ShapeDtypeStruct((L, N, 128), jnp.float32),
            jax.ShapeDtypeStruct((L, N, 32), jnp.float32),
            tab_shape, tab_shape,
        ],
    )(pe_emb, x,
      f['W'][:32], f['W'][32:], f['b'][None],
      pp['W'], pp['b'][None], pp['Wi'], pp['bi'][None],
      pp['Wo'], pp['bo'][None])


def _sc_body(tin_h, tout_h, sg_h, ss_h, dg_h, ds_h,
             zeros_h, dep_h, inc_h, outg_h,
             gidx_v, sidx_v, rows0_v, rows1_v, acc_sh,
             gsem0, gsem1, ssem0, ssem1):
    cid = lax.axis_index("c")
    sid = lax.axis_index("s")
    idx_row0 = cid * (IDX_ROWS // 2) + sid * SC_J
    fl0 = sid * 624
    flr = 640
    rows = (rows0_v, rows1_v)
    gsems = (gsem0, gsem1)
    ssems = (ssem0, ssem1)

    def gather(j, b, tab_h):
        pltpu.async_copy(tab_h.at[gidx_v.at[j]], rows[b], gsems[b])

    def scatter(j, b):
        pltpu.async_copy(rows[b], acc_sh.at[sidx_v.at[j]], ssems[b],
                         add=True)

    for tab_h, gh, sh, out_h in (
            (tin_h, sg_h, ds_h, inc_h),
            (tout_h, dg_h, ss_h, outg_h)):
        pltpu.sync_copy(zeros_h.at[pl.ds(fl0, flr)], acc_sh.at[pl.ds(fl0, flr)])
        pltpu.sync_copy(gh.at[pl.ds(idx_row0, SC_J)], gidx_v)
        pltpu.sync_copy(sh.at[pl.ds(idx_row0, SC_J)], sidx_v)
        plsc.subcore_barrier()

        gather(0, 0, tab_h)

        def jbody(i, carry):
            j0 = 2 * i
            j1 = 2 * i + 1
            pltpu.make_async_copy(tab_h.at[gidx_v.at[j0]], rows[0],
                                  gsems[0]).wait()
            scatter(j0, 0)

            @pl.when(i > 0)
            def _():
                pltpu.make_async_copy(rows[1], acc_sh.at[sidx_v.at[j1]],
                                      ssems[1]).wait()

            gather(j1, 1, tab_h)
            pltpu.make_async_copy(tab_h.at[gidx_v.at[j1]], rows[1],
                                  gsems[1]).wait()
            scatter(j1, 1)

            @pl.when(j1 + 1 < SC_J)
            def _():
                pltpu.make_async_copy(rows[0], acc_sh.at[sidx_v.at[j0]],
                                      ssems[0]).wait()
                gather(j1 + 1, 0, tab_h)

            return carry

        lax.fori_loop(0, SC_J // 2, jbody, 0)
        pltpu.make_async_copy(rows[0], acc_sh.at[sidx_v.at[0]],
                              ssems[0]).wait()
        pltpu.make_async_copy(rows[1], acc_sh.at[sidx_v.at[0]],
                              ssems[1]).wait()
        plsc.subcore_barrier()
        pltpu.sync_copy(acc_sh.at[pl.ds(fl0, flr)],
                        out_h.at[cid].at[pl.ds(fl0, flr)])
        plsc.subcore_barrier()


def _call_sc(tab_in, tab_out, sg, ss, dg, ds_, zeros_nd, dep):
    mesh = plsc.VectorSubcoreMesh(core_axis_name="c", subcore_axis_name="s")
    fn = pl.kernel(
        _sc_body,
        out_type=[jax.ShapeDtypeStruct((2, N, 128), jnp.float32)] * 2,
        mesh=mesh,
        scratch_types=[
            pltpu.VMEM((SC_J, SC_C), jnp.int32),
            pltpu.VMEM((SC_J, SC_C), jnp.int32),
            pltpu.VMEM((SC_C, 128), jnp.float32),
            pltpu.VMEM((SC_C, 128), jnp.float32),
            pltpu.VMEM_SHARED((N_ACC, 128), jnp.float32),
            pltpu.SemaphoreType.DMA,
            pltpu.SemaphoreType.DMA,
            pltpu.SemaphoreType.DMA,
            pltpu.SemaphoreType.DMA,
        ],
    )
    return fn(tab_in, tab_out, sg, ss, dg, ds_, zeros_nd, dep)


def _k2a_body(proj_ref, inc_ref, outg_ref,
              g1_ref, be1_ref, w2_ref, b2_ref, g2_ref, be2_ref,
              wk_ref, bk_ref, wv_ref,
              xn_ref, ktv_ref):
    n = pl.program_id(0)
    inc = inc_ref[0] + inc_ref[1]
    outg = outg_ref[0] + outg_ref[1]
    for l in range(L):
        agg_in = inc[:, l * 32:(l + 1) * 32]
        agg_out = outg[:, l * 32:(l + 1) * 32]
        out = jnp.concatenate([proj_ref[l], agg_in, agg_out], axis=-1)
        h1 = _ln(_silu(out), g1_ref[...], be1_ref[...])
        h2 = h1 @ w2_ref[...] + b2_ref[...]
        xn = _ln(_silu(h2), g2_ref[...], be2_ref[...])
        xn_ref[l] = xn

        k = _elu1(xn @ wk_ref[...] + bk_ref[...])
        v = xn @ wv_ref[...]
        ktv = lax.dot_general(k, v, (((0,), (0,)), ((), ())),
                              preferred_element_type=jnp.float32)

        @pl.when(n == 0)
        def _():
            ktv_ref[l] = ktv

        @pl.when(n != 0)
        def _():
            ktv_ref[l] += ktv


def _call_k2a(proj, incP, outgP, lp):
    f, ln_ = lp['feat'], lp['lin']
    part = pl.BlockSpec((2, BN, 128), lambda n: (0, n, 0))
    return pl.pallas_call(
        _k2a_body,
        grid=(NB,),
        in_specs=[
            _xspec(128), part, part,
            _full_spec((1, 192)), _full_spec((1, 192)),
            _full_spec((192, 128)), _full_spec((1, 128)),
            _full_spec((1, 128)), _full_spec((1, 128)),
            _full_spec((128, 64)), _full_spec((1, 64)),
            _full_spec((128, 32)),
        ],
        out_specs=[
            _xspec(128),
            pl.BlockSpec((L, 64, 32), lambda n: (0, 0, 0)),
        ],
        out_shape=[
            jax.ShapeDtypeStruct((L, N, 128), jnp.float32),
            jax.ShapeDtypeStruct((L, 64, 32), jnp.float32),
        ],
    )(proj, incP, outgP,
      f['g1'][None], f['be1'][None], f['W2'], f['b2'][None],
      f['g2'][None], f['be2'][None],
      ln_['Wk'], ln_['bk'][None], ln_['Wv'])


def _k2b_body(peproj_ref, pei_ref, peo_ref, pw2_ref, pb2_ref, pen_ref):
    pei = pei_ref[0] + pei_ref[1]
    peo = peo_ref[0] + peo_ref[1]
    for l in range(L):
        pe_in = pei[:, l * 32:(l + 1) * 32]
        pe_out = peo[:, l * 32:(l + 1) * 32]
        pout = jnp.concatenate([peproj_ref[l], pe_in, pe_out], axis=-1)
        pen_ref[l] = jnp.tanh(jnp.tanh(pout) @ pw2_ref[...] + pb2_ref[...])


def _call_k2b(peproj, peincP, peoutgP, lp):
    pp = lp['pe']
    part = pl.BlockSpec((2, BN, 128), lambda n: (0, n, 0))
    return pl.pallas_call(
        _k2b_body,
        grid=(NB,),
        in_specs=[
            _xspec(32), part, part,
            _full_spec((96, 32)), _full_spec((1, 32)),
        ],
        out_specs=_xspec(32),
        out_shape=jax.ShapeDtypeStruct((L, N, 32), jnp.float32),
    )(peproj, peincP, peoutgP, pp['W2'], pp['b2'][None])


def _k3_body(xn_ref, ktv_ref, wq_ref, bq_ref, wc_x_ref, wc_a_ref, bc_ref,
             x_ref):
    for l in range(L):
        xn = xn_ref[l]
        q = _elu1(xn @ wq_ref[...] + bq_ref[...])
        att = jnp.dot(q, ktv_ref[l], preferred_element_type=jnp.float32)
        x_ref[l] = _silu(xn @ wc_x_ref[...] + att @ wc_a_ref[...]
                         + bc_ref[...])


def _call_k3(x_new, ktv, lp):
    ln_, cb = lp['lin'], lp['comb']
    wc = cb['Wc']
    return pl.pallas_call(
        _k3_body,
        grid=(NB,),
        in_specs=[
            _xspec(128),
            pl.BlockSpec((L, 64, 32), lambda n: (0, 0, 0)),
            _full_spec((128, 64)), _full_spec((1, 64)),
            _full_spec((128, 128)), _full_spec((32, 128)),
            _full_spec((1, 128)),
        ],
        out_specs=_xspec(128),
        out_shape=jax.ShapeDtypeStruct((L, N, 128), jnp.float32),
    )(x_new, ktv, ln_['Wq'], ln_['bq'][None], wc[:128], wc[128:],
      cb['bc'][None])


def kernel(op_code, features, edge_index, lengths, params):
    p = params
    opc = op_code.astype(jnp.float32)[..., None]

    src, dst = edge_index[0], edge_index[1]
    pad = E_PAD - E
    zpad = jnp.arange(pad, dtype=jnp.int32) % 128
    npad = N + (jnp.arange(pad, dtype=jnp.int32) % 128)
    sg = jnp.concatenate([src, zpad]).reshape(IDX_ROWS, SC_C)
    ss = jnp.concatenate([src, npad]).reshape(IDX_ROWS, SC_C)
    dg = jnp.concatenate([dst, zpad]).reshape(IDX_ROWS, SC_C)
    ds_ = jnp.concatenate([dst, npad]).reshape(IDX_ROWS, SC_C)
    zeros_nd = jnp.zeros((N, 128), jnp.float32)

    x, pe_emb = _call_k0(opc, features, p)
    for lp in p['layers']:
        pin_t, pout_t = _call_k1a(pe_emb, x, lp)
        incP, outgP = _call_sc(pin_t, pout_t, sg, ss, dg, ds_, zeros_nd,
                               zeros_nd)
        proj, peproj, pepin_t, pepout_t = _call_k1b(pe_emb, x, lp)
        peincP, peoutgP = _call_sc(pepin_t, pepout_t, sg, ss, dg, ds_,
                                   zeros_nd, incP)
        x_new, ktv = _call_k2a(proj, incP, outgP, lp)
        x = _call_k3(x_new, ktv, lp)
        pe_emb = _call_k2b(peproj, peincP, peoutgP, lp)
    return x
